# baseline (speedup 1.0000x reference)
import functools

import jax
import jax.numpy as jnp
from jax import lax
from jax.experimental import pallas as pl
from jax.experimental.pallas import tpu as pltpu
from jax.experimental.pallas import tpu_sc as plsc

_BIG = float(3.0e38)


def _fps_b(xyz, npoint):
    nb, n, _ = xyz.shape
    cols = 128
    rows = max(1, -(-n // cols))
    rows8 = -(-rows // 8) * 8
    total = rows8 * cols
    pad = total - n
    if pad:
        xyz_p = jnp.concatenate(
            [xyz, jnp.broadcast_to(xyz[:, 0:1, :], (nb, pad, 3))], axis=1)
    else:
        xyz_p = xyz
    planes = xyz_p.transpose(0, 2, 1).reshape(nb, 3, rows8, cols)

    def body(planes_ref, rows_ref, *rest):
        out_refs = rest[:nb]
        dists_refs = rest[nb:]
        r_iota = lax.broadcasted_iota(jnp.int32, (rows8, cols), 0)
        c_iota = lax.broadcasted_iota(jnp.int32, (rows8, cols), 1)
        flat = r_iota * cols + c_iota
        for c in range(nb):
            dists_refs[c][...] = jnp.full((rows8, cols), 1e10, jnp.float32)

        def step(j, fars):
            new_fars = []
            for c in range(nb):
                far = fars[c]
                out_refs[c][0, j] = far
                crow = rows_ref[c, pl.ds(far, 1), :]
                cx = jnp.broadcast_to(crow[:, 0:1], (rows8, cols))
                cy = jnp.broadcast_to(crow[:, 1:2], (rows8, cols))
                cz = jnp.broadcast_to(crow[:, 2:3], (rows8, cols))
                dx = planes_ref[c, 0] - cx
                dy = planes_ref[c, 1] - cy
                dz = planes_ref[c, 2] - cz
                d = dx * dx + dy * dy + dz * dz
                nd = jnp.minimum(dists_refs[c][...], d)
                dists_refs[c][...] = nd
                mx = jnp.max(nd)
                new_fars.append(
                    jnp.min(jnp.where(nd == mx, flat, total)).astype(jnp.int32))
            return tuple(new_fars)

        lax.fori_loop(0, npoint, step, tuple(jnp.int32(0) for _ in range(nb)))

    outs = pl.pallas_call(
        body,
        in_specs=[
            pl.BlockSpec(memory_space=pltpu.VMEM),
            pl.BlockSpec(memory_space=pltpu.VMEM),
        ],
        out_specs=[pl.BlockSpec(memory_space=pltpu.SMEM)] * nb,
        out_shape=[jax.ShapeDtypeStruct((1, npoint), jnp.int32)] * nb,
        scratch_shapes=[pltpu.VMEM((rows8, cols), jnp.float32)] * nb,
    )(planes, xyz_p)
    return jnp.concatenate(outs, axis=0)


def _knn_b(query, points, k):
    nb, m, _ = query.shape
    n = points.shape[1]
    bm = min(m, 64)
    qp = jnp.pad(query, ((0, 0), (0, 0), (0, 5)))
    dt = jnp.pad(points.transpose(0, 2, 1), ((0, 0), (0, 5), (0, 0)))

    def body(q_ref, dt_ref, idx_ref, d_ref):
        q = q_ref[0]
        dtm = dt_ref[0]
        qs = jnp.sum(q * q, axis=1, keepdims=True)
        ps = jnp.sum(dtm * dtm, axis=0, keepdims=True)
        prod = lax.dot_general(q, dtm, (((1,), (0,)), ((), ())),
                               preferred_element_type=jnp.float32)
        cur = (-2.0 * prod + qs) + ps
        lane = lax.broadcasted_iota(jnp.int32, (bm, n), 1)
        idx_cols, d_cols = [], []
        for _ in range(k):
            dmin = jnp.min(cur, axis=1, keepdims=True)
            sel = cur == dmin
            ij = jnp.min(jnp.where(sel, lane, n), axis=1, keepdims=True)
            idx_cols.append(ij)
            d_cols.append(dmin)
            cur = jnp.where(lane == ij, _BIG, cur)
        idx_ref[0] = jnp.concatenate(idx_cols, axis=1)
        d_ref[0] = jnp.concatenate(d_cols, axis=1)

    idx, d = pl.pallas_call(
        body,
        grid=(nb, m // bm),
        in_specs=[
            pl.BlockSpec((1, bm, 8), lambda b, i: (b, i, 0)),
            pl.BlockSpec((1, 8, n), lambda b, i: (b, 0, 0)),
        ],
        out_specs=[
            pl.BlockSpec((1, bm, k), lambda b, i: (b, i, 0)),
            pl.BlockSpec((1, bm, k), lambda b, i: (b, i, 0)),
        ],
        out_shape=[
            jax.ShapeDtypeStruct((nb, m, k), jnp.int32),
            jax.ShapeDtypeStruct((nb, m, k), jnp.float32),
        ],
    )(qp, dt)
    return idx, d


def _sc_gather(table, idx):
    V, D = table.shape
    Bi = idx.shape[0]
    info = plsc.get_sparse_core_info()
    NC, NS = info.num_cores, info.num_subcores
    NW = NC * NS
    b_per_w = Bi // NW
    CH = min(b_per_w, 128)
    n_ch = b_per_w // CH
    mesh = plsc.VectorSubcoreMesh(core_axis_name="c", subcore_axis_name="s")

    @functools.partial(
        pl.kernel, mesh=mesh,
        compiler_params=pltpu.CompilerParams(use_tc_tiling_on_sc=False),
        out_type=jax.ShapeDtypeStruct((Bi, D), jnp.float32),
        scratch_types=[
            pltpu.VMEM((b_per_w,), jnp.int32),
            pltpu.VMEM((CH, D), jnp.float32),
            pltpu.SemaphoreType.DMA,
        ],
    )
    def gk(table_hbm, idx_hbm, out_hbm, idx_v, rows_v, sem):
        wid = lax.axis_index("s") * NC + lax.axis_index("c")
        base = wid * b_per_w
        pltpu.sync_copy(idx_hbm.at[pl.ds(base, b_per_w)], idx_v)

        def chunk(i, carry):
            pltpu.async_copy(table_hbm.at[idx_v.at[pl.ds(i * CH, CH)]],
                             rows_v, sem).wait()
            pltpu.sync_copy(rows_v, out_hbm.at[pl.ds(base + i * CH, CH)])
            return carry

        lax.fori_loop(0, n_ch, chunk, jnp.int32(0))

    return gk(table, idx)


def _gather_rows(table, idx):
    V, D = table.shape
    Dp = -(-D // 16) * 16
    if Dp != D:
        table = jnp.pad(table, ((0, 0), (0, Dp - D)))
    Bi = idx.shape[0]
    Bp = -(-Bi // 256) * 256
    idx_p = jnp.pad(idx, (0, Bp - Bi)) if Bp != Bi else idx
    rows = _sc_gather(table, idx_p.astype(jnp.int32))
    return rows[:Bi, :D]


def _mlp(x3, layers, pool, prep=None, extras=(), cprep=None):
    k, mp, cin = x3.shape
    cw = cprep if cprep is not None else cin
    cout = layers[-1][0].shape[1] if layers else cw
    gm = min(mp, 512)
    while gm > 8 and k * gm * max(cin, cw, cout) * 4 > 4 * 1024 * 1024:
        gm //= 2
    while mp % gm:
        gm //= 2
    ops = [x3]
    in_specs = [pl.BlockSpec((k, gm, cin), lambda i: (0, i, 0))]
    for e in extras:
        ops.append(e)
        ce = e.shape[1]
        in_specs.append(pl.BlockSpec((gm, ce), lambda i: (i, 0)))
    for (W, b, _r) in layers:
        ops.append(W)
        in_specs.append(pl.BlockSpec(W.shape, lambda i: (0, 0)))
        if b is not None:
            ops.append(b.reshape(1, -1))
            in_specs.append(pl.BlockSpec((1, b.size), lambda i: (0, 0)))
    ne = len(extras)

    def body(*refs):
        x_ref, o_ref = refs[0], refs[-1]
        e_vals = [r[...] for r in refs[1:1 + ne]]
        w_refs = refs[1 + ne:-1]

        def chain(x):
            wi = 0
            for (W, b, relu) in layers:
                x = lax.dot_general(x, w_refs[wi][...],
                                    (((1,), (0,)), ((), ())),
                                    preferred_element_type=jnp.float32)
                wi += 1
                if b is not None:
                    x = x + w_refs[wi][...]
                    wi += 1
                if relu:
                    x = jnp.maximum(x, 0.0)
            return x

        def make_x(j):
            xr = x_ref[j]
            return prep(xr, *e_vals) if prep is not None else xr

        if pool == 'max':
            def jstep(j, acc):
                return jnp.maximum(acc, chain(make_x(j)))
            o_ref[...] = lax.fori_loop(0, k, jstep,
                                       jnp.full((gm, cout), -_BIG, jnp.float32))
        elif pool == 'interp3':
            d_v, f1_v = e_vals
            dd = jnp.maximum(d_v, 1e-10)
            w = 1.0 / dd
            w = w / jnp.sum(w, axis=1, keepdims=True)

            def wj(j):
                return jnp.broadcast_to(w[:, j:j + 1], (gm, cin))

            xi = (x_ref[0] * wj(0) + x_ref[1] * wj(1)) + x_ref[2] * wj(2)
            o_ref[...] = chain(jnp.concatenate([xi, f1_v], axis=1))
        else:
            o_ref[...] = chain(make_x(0))

    return pl.pallas_call(
        body,
        grid=(mp // gm,),
        in_specs=in_specs,
        out_specs=pl.BlockSpec((gm, cout), lambda i: (i, 0)),
        out_shape=jax.ShapeDtypeStruct((mp, cout), jnp.float32),
    )(*ops)


def _offs(nb, n):
    return (jnp.arange(nb, dtype=jnp.int32) * n)[:, None, None]


def _grouped_rows(points, feats, idx):
    nb, n, _ = points.shape
    c = feats.shape[-1]
    k = idx.shape[-1]
    m = idx.shape[1]
    table = jnp.concatenate([points, feats], -1).reshape(nb * n, 3 + c)
    idx_f = jnp.transpose(idx + _offs(nb, n), (2, 0, 1)).reshape(-1)
    return _gather_rows(table, idx_f).reshape(k, nb * m, 3 + c)


def _sa(xyz, feat, npoint, k, Ws):
    nb, n, _ = xyz.shape
    if npoint < n:
        fidx = _fps_b(xyz, npoint)
        tab = xyz.reshape(nb * n, 3)
        gidx = (fidx + jnp.arange(nb, dtype=jnp.int32)[:, None] * n).reshape(-1)
        new_xyz = _gather_rows(tab, gidx).reshape(nb, npoint, 3)
    else:
        new_xyz = xyz
    idx, _ = _knn_b(new_xyz, xyz, k)
    rows = _grouped_rows(xyz, feat, idx)
    q = new_xyz.reshape(nb * npoint, 3)

    def prep(xr, qb):
        return jnp.concatenate([xr[:, :3] - qb, xr[:, 3:]], axis=1)

    out = _mlp(rows, [(W, None, True) for W in Ws], 'max',
               prep=prep, extras=(q,))
    return new_xyz, out.reshape(nb, npoint, -1)


def _flow_embedding(p1, p2, f1, f2, k, Ws):
    nb, m, _ = p1.shape
    idx, _ = _knn_b(p1, p2, k)
    rows = _grouped_rows(p2, f2, idx)
    q = p1.reshape(nb * m, 3)
    f1r = f1.reshape(nb * m, -1)
    c2 = f2.shape[-1]
    c1 = f1r.shape[-1]

    def prep(xr, qb, f1b):
        return jnp.concatenate([xr[:, 3:], f1b, xr[:, :3] - qb], axis=1)

    out = _mlp(rows, [(W, None, True) for W in Ws], 'max',
               prep=prep, extras=(q, f1r), cprep=c2 + c1 + 3)
    return out.reshape(nb, m, -1)


def _set_upconv(p1, p2, f1, f2, k, mlp_w, mlp2_w):
    nb, m, _ = p1.shape
    idx, _ = _knn_b(p1, p2, k)
    rows = _grouped_rows(p2, f2, idx)
    q = p1.reshape(nb * m, 3)

    def prep(xr, qb):
        return jnp.concatenate([xr[:, 3:], xr[:, :3] - qb], axis=1)

    pooled = _mlp(rows, [(W, None, True) for W in mlp_w], 'max',
                  prep=prep, extras=(q,))
    f1r = f1.reshape(nb * m, -1)

    def prep2(xr, f1b):
        return jnp.concatenate([xr, f1b], axis=1)

    out = _mlp(pooled[None], [(W, None, True) for W in mlp2_w], 'none',
               prep=prep2, extras=(f1r,),
               cprep=pooled.shape[-1] + f1r.shape[-1])
    return out.reshape(nb, m, -1)


def _feature_prop(p1, p2, f1, f2, Ws):
    nb, m, _ = p1.shape
    n = p2.shape[1]
    c = f2.shape[-1]
    idx, d = _knn_b(p1, p2, 3)
    idx_f = jnp.transpose(idx + _offs(nb, n), (2, 0, 1)).reshape(-1)
    rows = _gather_rows(f2.reshape(nb * n, c), idx_f).reshape(3, nb * m, c)
    f1r = f1.reshape(nb * m, -1)
    out = _mlp(rows, [(W, None, True) for W in Ws], 'interp3',
               extras=(d.reshape(nb * m, 3), f1r),
               cprep=c + f1r.shape[-1])
    return out.reshape(nb, m, -1)


def kernel(pc1, pc2, feature1, feature2, params):
    P = params
    x1 = pc1.transpose(0, 2, 1)
    x2 = pc2.transpose(0, 2, 1)
    ft1 = feature1.transpose(0, 2, 1)
    ft2 = feature2.transpose(0, 2, 1)

    xyz0 = jnp.concatenate([x1, x2], 0)
    feat0 = jnp.concatenate([ft1, ft2], 0)

    l0p, l0f = _sa(xyz0, feat0, 2048, 16, P['sa0'])
    l1p, l1f = _sa(l0p, l0f, 2048, 16, P['sa1'])
    l2p, l2f = _sa(l1p, l1f, 512, 16, P['sa2'])

    l2p1, l2p2 = l2p[:2], l2p[2:]
    l2f1, l2f2 = l2f[:2], l2f[2:]
    l1p1, l1f1 = l1p[:2], l1f[:2]

    l2f1n = _flow_embedding(l2p1, l2p2, l2f1, l2f2, 64, P['fe'])

    l3p1, l3f1 = _sa(l2p1, l2f1n, 128, 8, P['sa3'])
    l4p1, l4f1 = _sa(l3p1, l3f1, 32, 8, P['sa4'])

    l3fn = _set_upconv(l3p1, l4p1, l3f1, l4f1, 8, [], P['su1_mlp2'])
    l2fn = _set_upconv(l2p1, l3p1,
                       jnp.concatenate([l2f1, l2f1n], -1), l3fn, 8,
                       P['su2_mlp'], P['su2_mlp2'])
    l1fn = _set_upconv(l1p1, l2p1, l1f1, l2fn, 8, P['su3_mlp'], P['su3_mlp2'])

    l0fn = _feature_prop(x1, l1p1, ft1, l1fn, P['fp'])

    out = _mlp(l0fn.reshape(1, 2 * 8192, 256),
               [(P['conv1'], None, True),
                (P['conv2_w'], P['conv2_b'], False)], 'none')
    return out.reshape(2, 8192, 3)


# speedup vs baseline: 1.3291x; 1.3291x over previous
import functools

import jax
import jax.numpy as jnp
from jax import lax
from jax.experimental import pallas as pl
from jax.experimental.pallas import tpu as pltpu
from jax.experimental.pallas import tpu_sc as plsc

_BIG = float(3.0e38)


def _fps_b(xyz, npoint):
    nb, n, _ = xyz.shape
    cols = 128
    rows = max(1, -(-n // cols))
    rows8 = -(-rows // 8) * 8
    total = rows8 * cols
    pad = total - n
    if pad:
        xyz_p = jnp.concatenate(
            [xyz, jnp.broadcast_to(xyz[:, 0:1, :], (nb, pad, 3))], axis=1)
    else:
        xyz_p = xyz
    planes = xyz_p.transpose(0, 2, 1).reshape(nb, 3, rows8, cols)

    planes3 = jnp.transpose(xyz_p, (2, 0, 1)).reshape(3, nb, rows8, cols)

    def body(planes_ref, rows_ref, *rest):
        out_refs = rest[:nb]
        dists_ref = rest[nb]
        r_iota = lax.broadcasted_iota(jnp.int32, (nb, rows8, cols), 1)
        c_iota = lax.broadcasted_iota(jnp.int32, (nb, rows8, cols), 2)
        flat = r_iota * cols + c_iota
        dists_ref[...] = jnp.full((nb, rows8, cols), 1e10, jnp.float32)

        def step(j, fars):
            crows = [rows_ref[c, pl.ds(fars[c], 1), :] for c in range(nb)]
            for c in range(nb):
                out_refs[c][0, j] = fars[c]
            cxyz = jnp.concatenate(crows, axis=0)

            def cplane(a):
                return jnp.broadcast_to(cxyz[:, a][:, None, None],
                                        (nb, rows8, cols))

            dx = planes_ref[0] - cplane(0)
            dy = planes_ref[1] - cplane(1)
            dz = planes_ref[2] - cplane(2)
            d = dx * dx + dy * dy + dz * dz
            nd = jnp.minimum(dists_ref[...], d)
            dists_ref[...] = nd
            mx = jnp.max(nd, axis=(1, 2), keepdims=True)
            fidx = jnp.min(jnp.where(nd == mx, flat, total),
                           axis=(1, 2))
            nbi = lax.broadcasted_iota(jnp.int32, (nb,), 0)
            return tuple(
                jnp.min(jnp.where(nbi == c, fidx, total)).astype(jnp.int32)
                for c in range(nb))

        lax.fori_loop(0, npoint, step, tuple(jnp.int32(0) for _ in range(nb)))

    outs = pl.pallas_call(
        body,
        in_specs=[
            pl.BlockSpec(memory_space=pltpu.VMEM),
            pl.BlockSpec(memory_space=pltpu.VMEM),
        ],
        out_specs=[pl.BlockSpec(memory_space=pltpu.SMEM)] * nb,
        out_shape=[jax.ShapeDtypeStruct((1, npoint), jnp.int32)] * nb,
        scratch_shapes=[pltpu.VMEM((nb, rows8, cols), jnp.float32)],
    )(planes3, xyz_p)
    return jnp.concatenate(outs, axis=0)


def _knn_b(query, points, k):
    nb, m, _ = query.shape
    n = points.shape[1]
    bm = min(m, 64)
    qp = jnp.pad(query, ((0, 0), (0, 0), (0, 5)))
    dt = jnp.pad(points.transpose(0, 2, 1), ((0, 0), (0, 5), (0, 0)))

    def body(q_ref, dt_ref, idx_ref, d_ref):
        q = q_ref[0]
        dtm = dt_ref[0]
        qs = jnp.sum(q * q, axis=1, keepdims=True)
        ps = jnp.sum(dtm * dtm, axis=0, keepdims=True)
        prod = lax.dot_general(q, dtm, (((1,), (0,)), ((), ())),
                               preferred_element_type=jnp.float32)
        cur = (-2.0 * prod + qs) + ps
        lane = lax.broadcasted_iota(jnp.int32, (bm, n), 1)
        idx_cols, d_cols = [], []
        for _ in range(k):
            dmin = jnp.min(cur, axis=1, keepdims=True)
            sel = cur == dmin
            ij = jnp.min(jnp.where(sel, lane, n), axis=1, keepdims=True)
            idx_cols.append(ij)
            d_cols.append(dmin)
            cur = jnp.where(lane == ij, _BIG, cur)
        idx_ref[0] = jnp.concatenate(idx_cols, axis=1)
        d_ref[0] = jnp.concatenate(d_cols, axis=1)

    idx, d = pl.pallas_call(
        body,
        grid=(nb, m // bm),
        in_specs=[
            pl.BlockSpec((1, bm, 8), lambda b, i: (b, i, 0)),
            pl.BlockSpec((1, 8, n), lambda b, i: (b, 0, 0)),
        ],
        out_specs=[
            pl.BlockSpec((1, bm, k), lambda b, i: (b, i, 0)),
            pl.BlockSpec((1, bm, k), lambda b, i: (b, i, 0)),
        ],
        out_shape=[
            jax.ShapeDtypeStruct((nb, m, k), jnp.int32),
            jax.ShapeDtypeStruct((nb, m, k), jnp.float32),
        ],
    )(qp, dt)
    return idx, d


def _sc_gather(table, idx):
    V, D = table.shape
    Bi = idx.shape[0]
    info = plsc.get_sparse_core_info()
    NC, NS = info.num_cores, info.num_subcores
    NW = NC * NS
    b_per_w = Bi // NW
    CH = min(b_per_w, 128)
    n_ch = b_per_w // CH
    mesh = plsc.VectorSubcoreMesh(core_axis_name="c", subcore_axis_name="s")

    @functools.partial(
        pl.kernel, mesh=mesh,
        compiler_params=pltpu.CompilerParams(use_tc_tiling_on_sc=False),
        out_type=jax.ShapeDtypeStruct((Bi, D), jnp.float32),
        scratch_types=[
            pltpu.VMEM((b_per_w,), jnp.int32),
            pltpu.VMEM((CH, D), jnp.float32),
            pltpu.SemaphoreType.DMA,
        ],
    )
    def gk(table_hbm, idx_hbm, out_hbm, idx_v, rows_v, sem):
        wid = lax.axis_index("s") * NC + lax.axis_index("c")
        base = wid * b_per_w
        pltpu.sync_copy(idx_hbm.at[pl.ds(base, b_per_w)], idx_v)

        def chunk(i, carry):
            pltpu.async_copy(table_hbm.at[idx_v.at[pl.ds(i * CH, CH)]],
                             rows_v, sem).wait()
            pltpu.sync_copy(rows_v, out_hbm.at[pl.ds(base + i * CH, CH)])
            return carry

        lax.fori_loop(0, n_ch, chunk, jnp.int32(0))

    return gk(table, idx)


def _gather_rows(table, idx):
    V, D = table.shape
    Dp = -(-D // 16) * 16
    if Dp != D:
        table = jnp.pad(table, ((0, 0), (0, Dp - D)))
    Bi = idx.shape[0]
    Bp = -(-Bi // 256) * 256
    idx_p = jnp.pad(idx, (0, Bp - Bi)) if Bp != Bi else idx
    rows = _sc_gather(table, idx_p.astype(jnp.int32))
    return rows[:Bi, :D]


def _mlp(x3, layers, pool, prep=None, extras=(), cprep=None):
    k, mp, cin = x3.shape
    cw = cprep if cprep is not None else cin
    cout = layers[-1][0].shape[1] if layers else cw
    gm = min(mp, 512)
    while gm > 8 and k * gm * max(cin, cw, cout) * 4 > 4 * 1024 * 1024:
        gm //= 2
    while mp % gm:
        gm //= 2
    ops = [x3]
    in_specs = [pl.BlockSpec((k, gm, cin), lambda i: (0, i, 0))]
    for e in extras:
        ops.append(e)
        ce = e.shape[1]
        in_specs.append(pl.BlockSpec((gm, ce), lambda i: (i, 0)))
    for (W, b, _r) in layers:
        ops.append(W)
        in_specs.append(pl.BlockSpec(W.shape, lambda i: (0, 0)))
        if b is not None:
            ops.append(b.reshape(1, -1))
            in_specs.append(pl.BlockSpec((1, b.size), lambda i: (0, 0)))
    ne = len(extras)

    def body(*refs):
        x_ref, o_ref = refs[0], refs[-1]
        e_vals = [r[...] for r in refs[1:1 + ne]]
        w_refs = refs[1 + ne:-1]

        def chain(x):
            wi = 0
            for (W, b, relu) in layers:
                x = lax.dot_general(x, w_refs[wi][...],
                                    (((1,), (0,)), ((), ())),
                                    preferred_element_type=jnp.float32)
                wi += 1
                if b is not None:
                    x = x + w_refs[wi][...]
                    wi += 1
                if relu:
                    x = jnp.maximum(x, 0.0)
            return x

        def make_x(j):
            xr = x_ref[j]
            return prep(xr, *e_vals) if prep is not None else xr

        if pool == 'max':
            def jstep(j, acc):
                return jnp.maximum(acc, chain(make_x(j)))
            o_ref[...] = lax.fori_loop(0, k, jstep,
                                       jnp.full((gm, cout), -_BIG, jnp.float32))
        elif pool == 'interp3':
            d_v, f1_v = e_vals
            dd = jnp.maximum(d_v, 1e-10)
            w = 1.0 / dd
            w = w / jnp.sum(w, axis=1, keepdims=True)

            def wj(j):
                return jnp.broadcast_to(w[:, j:j + 1], (gm, cin))

            xi = (x_ref[0] * wj(0) + x_ref[1] * wj(1)) + x_ref[2] * wj(2)
            o_ref[...] = chain(jnp.concatenate([xi, f1_v], axis=1))
        else:
            o_ref[...] = chain(make_x(0))

    return pl.pallas_call(
        body,
        grid=(mp // gm,),
        in_specs=in_specs,
        out_specs=pl.BlockSpec((gm, cout), lambda i: (i, 0)),
        out_shape=jax.ShapeDtypeStruct((mp, cout), jnp.float32),
    )(*ops)


def _offs(nb, n):
    return (jnp.arange(nb, dtype=jnp.int32) * n)[:, None, None]


def _grouped_rows(points, feats, idx):
    nb, n, _ = points.shape
    c = feats.shape[-1]
    k = idx.shape[-1]
    m = idx.shape[1]
    table = jnp.concatenate([points, feats], -1).reshape(nb * n, 3 + c)
    idx_f = jnp.transpose(idx + _offs(nb, n), (2, 0, 1)).reshape(-1)
    return _gather_rows(table, idx_f).reshape(k, nb * m, 3 + c)


def _sa(xyz, feat, npoint, k, Ws):
    nb, n, _ = xyz.shape
    if npoint < n:
        fidx = _fps_b(xyz, npoint)
        tab = xyz.reshape(nb * n, 3)
        gidx = (fidx + jnp.arange(nb, dtype=jnp.int32)[:, None] * n).reshape(-1)
        new_xyz = _gather_rows(tab, gidx).reshape(nb, npoint, 3)
    else:
        new_xyz = xyz
    idx, _ = _knn_b(new_xyz, xyz, k)
    rows = _grouped_rows(xyz, feat, idx)
    q = new_xyz.reshape(nb * npoint, 3)

    def prep(xr, qb):
        return jnp.concatenate([xr[:, :3] - qb, xr[:, 3:]], axis=1)

    out = _mlp(rows, [(W, None, True) for W in Ws], 'max',
               prep=prep, extras=(q,))
    return new_xyz, out.reshape(nb, npoint, -1)


def _flow_embedding(p1, p2, f1, f2, k, Ws):
    nb, m, _ = p1.shape
    idx, _ = _knn_b(p1, p2, k)
    rows = _grouped_rows(p2, f2, idx)
    q = p1.reshape(nb * m, 3)
    f1r = f1.reshape(nb * m, -1)
    c2 = f2.shape[-1]
    c1 = f1r.shape[-1]

    def prep(xr, qb, f1b):
        return jnp.concatenate([xr[:, 3:], f1b, xr[:, :3] - qb], axis=1)

    out = _mlp(rows, [(W, None, True) for W in Ws], 'max',
               prep=prep, extras=(q, f1r), cprep=c2 + c1 + 3)
    return out.reshape(nb, m, -1)


def _set_upconv(p1, p2, f1, f2, k, mlp_w, mlp2_w):
    nb, m, _ = p1.shape
    idx, _ = _knn_b(p1, p2, k)
    rows = _grouped_rows(p2, f2, idx)
    q = p1.reshape(nb * m, 3)

    def prep(xr, qb):
        return jnp.concatenate([xr[:, 3:], xr[:, :3] - qb], axis=1)

    pooled = _mlp(rows, [(W, None, True) for W in mlp_w], 'max',
                  prep=prep, extras=(q,))
    f1r = f1.reshape(nb * m, -1)

    def prep2(xr, f1b):
        return jnp.concatenate([xr, f1b], axis=1)

    out = _mlp(pooled[None], [(W, None, True) for W in mlp2_w], 'none',
               prep=prep2, extras=(f1r,),
               cprep=pooled.shape[-1] + f1r.shape[-1])
    return out.reshape(nb, m, -1)


def _feature_prop(p1, p2, f1, f2, Ws):
    nb, m, _ = p1.shape
    n = p2.shape[1]
    c = f2.shape[-1]
    idx, d = _knn_b(p1, p2, 3)
    idx_f = jnp.transpose(idx + _offs(nb, n), (2, 0, 1)).reshape(-1)
    rows = _gather_rows(f2.reshape(nb * n, c), idx_f).reshape(3, nb * m, c)
    f1r = f1.reshape(nb * m, -1)
    out = _mlp(rows, [(W, None, True) for W in Ws], 'interp3',
               extras=(d.reshape(nb * m, 3), f1r),
               cprep=c + f1r.shape[-1])
    return out.reshape(nb, m, -1)


def kernel(pc1, pc2, feature1, feature2, params):
    P = params
    x1 = pc1.transpose(0, 2, 1)
    x2 = pc2.transpose(0, 2, 1)
    ft1 = feature1.transpose(0, 2, 1)
    ft2 = feature2.transpose(0, 2, 1)

    xyz0 = jnp.concatenate([x1, x2], 0)
    feat0 = jnp.concatenate([ft1, ft2], 0)

    l0p, l0f = _sa(xyz0, feat0, 2048, 16, P['sa0'])
    l1p, l1f = _sa(l0p, l0f, 2048, 16, P['sa1'])
    l2p, l2f = _sa(l1p, l1f, 512, 16, P['sa2'])

    l2p1, l2p2 = l2p[:2], l2p[2:]
    l2f1, l2f2 = l2f[:2], l2f[2:]
    l1p1, l1f1 = l1p[:2], l1f[:2]

    l2f1n = _flow_embedding(l2p1, l2p2, l2f1, l2f2, 64, P['fe'])

    l3p1, l3f1 = _sa(l2p1, l2f1n, 128, 8, P['sa3'])
    l4p1, l4f1 = _sa(l3p1, l3f1, 32, 8, P['sa4'])

    l3fn = _set_upconv(l3p1, l4p1, l3f1, l4f1, 8, [], P['su1_mlp2'])
    l2fn = _set_upconv(l2p1, l3p1,
                       jnp.concatenate([l2f1, l2f1n], -1), l3fn, 8,
                       P['su2_mlp'], P['su2_mlp2'])
    l1fn = _set_upconv(l1p1, l2p1, l1f1, l2fn, 8, P['su3_mlp'], P['su3_mlp2'])

    l0fn = _feature_prop(x1, l1p1, ft1, l1fn, P['fp'])

    out = _mlp(l0fn.reshape(1, 2 * 8192, 256),
               [(P['conv1'], None, True),
                (P['conv2_w'], P['conv2_b'], False)], 'none')
    return out.reshape(2, 8192, 3)


# speedup vs baseline: 1.3310x; 1.0014x over previous
import functools

import jax
import jax.numpy as jnp
from jax import lax
from jax.experimental import pallas as pl
from jax.experimental.pallas import tpu as pltpu
from jax.experimental.pallas import tpu_sc as plsc

_BIG = float(3.0e38)


def _fps_b(xyz, npoint):
    nb, n, _ = xyz.shape
    cols = 128
    rows = max(1, -(-n // cols))
    rows8 = -(-rows // 8) * 8
    total = rows8 * cols
    pad = total - n
    if pad:
        xyz_p = jnp.concatenate(
            [xyz, jnp.broadcast_to(xyz[:, 0:1, :], (nb, pad, 3))], axis=1)
    else:
        xyz_p = xyz
    planes = xyz_p.transpose(0, 2, 1).reshape(nb, 3, rows8, cols)

    planes3 = jnp.transpose(xyz_p, (2, 0, 1)).reshape(3, nb, rows8, cols)

    def body(planes_ref, rows_ref, *rest):
        out_refs = rest[:nb]
        dists_ref = rest[nb]
        r_iota = lax.broadcasted_iota(jnp.int32, (nb, rows8, cols), 1)
        c_iota = lax.broadcasted_iota(jnp.int32, (nb, rows8, cols), 2)
        flat = r_iota * cols + c_iota
        dists_ref[...] = jnp.full((nb, rows8, cols), 1e10, jnp.float32)

        def step(j, fars):
            crows = [rows_ref[c, pl.ds(fars[c], 1), :] for c in range(nb)]
            for c in range(nb):
                out_refs[c][0, j] = fars[c]
            cxyz = jnp.concatenate(crows, axis=0)

            def cplane(a):
                return jnp.broadcast_to(cxyz[:, a][:, None, None],
                                        (nb, rows8, cols))

            dx = planes_ref[0] - cplane(0)
            dy = planes_ref[1] - cplane(1)
            dz = planes_ref[2] - cplane(2)
            d = dx * dx + dy * dy + dz * dz
            nd = jnp.minimum(dists_ref[...], d)
            dists_ref[...] = nd
            mx = jnp.max(nd, axis=(1, 2), keepdims=True)
            fidx = jnp.min(jnp.where(nd == mx, flat, total),
                           axis=(1, 2))
            nbi = lax.broadcasted_iota(jnp.int32, (nb,), 0)
            return tuple(
                jnp.min(jnp.where(nbi == c, fidx, total)).astype(jnp.int32)
                for c in range(nb))

        lax.fori_loop(0, npoint, step, tuple(jnp.int32(0) for _ in range(nb)))

    outs = pl.pallas_call(
        body,
        in_specs=[
            pl.BlockSpec(memory_space=pltpu.VMEM),
            pl.BlockSpec(memory_space=pltpu.VMEM),
        ],
        out_specs=[pl.BlockSpec(memory_space=pltpu.SMEM)] * nb,
        out_shape=[jax.ShapeDtypeStruct((1, npoint), jnp.int32)] * nb,
        scratch_shapes=[pltpu.VMEM((nb, rows8, cols), jnp.float32)],
    )(planes3, xyz_p)
    return jnp.concatenate(outs, axis=0)


def _knn_b(query, points, k):
    nb, m, _ = query.shape
    n = points.shape[1]
    bm = min(m, 64)
    qp = jnp.pad(query, ((0, 0), (0, 0), (0, 5)))
    dt = jnp.pad(points.transpose(0, 2, 1), ((0, 0), (0, 5), (0, 0)))

    def body(q_ref, dt_ref, idx_ref, d_ref):
        q = q_ref[0]
        dtm = dt_ref[0]
        qs = jnp.sum(q * q, axis=1, keepdims=True)
        ps = jnp.sum(dtm * dtm, axis=0, keepdims=True)
        prod = lax.dot_general(q, dtm, (((1,), (0,)), ((), ())),
                               preferred_element_type=jnp.float32)
        cur = (-2.0 * prod + qs) + ps
        lane = lax.broadcasted_iota(jnp.int32, (bm, n), 1)
        idx_cols, d_cols = [], []
        for _ in range(k):
            dmin = jnp.min(cur, axis=1, keepdims=True)
            sel = cur == dmin
            ij = jnp.min(jnp.where(sel, lane, n), axis=1, keepdims=True)
            idx_cols.append(ij)
            d_cols.append(dmin)
            cur = jnp.where(lane == ij, _BIG, cur)
        idx_ref[0] = jnp.concatenate(idx_cols, axis=1)
        d_ref[0] = jnp.concatenate(d_cols, axis=1)

    idx, d = pl.pallas_call(
        body,
        grid=(nb, m // bm),
        in_specs=[
            pl.BlockSpec((1, bm, 8), lambda b, i: (b, i, 0)),
            pl.BlockSpec((1, 8, n), lambda b, i: (b, 0, 0)),
        ],
        out_specs=[
            pl.BlockSpec((1, bm, k), lambda b, i: (b, i, 0)),
            pl.BlockSpec((1, bm, k), lambda b, i: (b, i, 0)),
        ],
        out_shape=[
            jax.ShapeDtypeStruct((nb, m, k), jnp.int32),
            jax.ShapeDtypeStruct((nb, m, k), jnp.float32),
        ],
    )(qp, dt)
    return idx, d


def _sc_gather(table, idx):
    V, D = table.shape
    Bi = idx.shape[0]
    info = plsc.get_sparse_core_info()
    NC, NS = info.num_cores, info.num_subcores
    NW = NC * NS
    b_per_w = Bi // NW
    CH = min(b_per_w, 128)
    n_ch = b_per_w // CH
    mesh = plsc.VectorSubcoreMesh(core_axis_name="c", subcore_axis_name="s")

    @functools.partial(
        pl.kernel, mesh=mesh,
        compiler_params=pltpu.CompilerParams(use_tc_tiling_on_sc=False),
        out_type=jax.ShapeDtypeStruct((Bi, D), jnp.float32),
        scratch_types=[
            pltpu.VMEM((b_per_w,), jnp.int32),
            pltpu.VMEM((CH, D), jnp.float32),
            pltpu.VMEM((CH, D), jnp.float32),
            pltpu.SemaphoreType.DMA,
            pltpu.SemaphoreType.DMA,
            pltpu.SemaphoreType.DMA,
            pltpu.SemaphoreType.DMA,
        ],
    )
    def gk(table_hbm, idx_hbm, out_hbm, idx_v, rv0, rv1, gs0, gs1, ss0, ss1):
        wid = lax.axis_index("s") * NC + lax.axis_index("c")
        base = wid * b_per_w
        pltpu.sync_copy(idx_hbm.at[pl.ds(base, b_per_w)], idx_v)
        bufs, gsems, ssems = (rv0, rv1), (gs0, gs1), (ss0, ss1)

        def gather(i):
            return pltpu.async_copy(
                table_hbm.at[idx_v.at[pl.ds(i * CH, CH)]],
                bufs[i % 2], gsems[i % 2])

        def scatter(i):
            return pltpu.async_copy(
                bufs[i % 2], out_hbm.at[pl.ds(base + i * CH, CH)],
                ssems[i % 2])

        cps, scs = {}, {}
        cps[0] = gather(0)
        for i in range(n_ch):
            if i + 1 < n_ch:
                if i >= 1:
                    scs[i - 1].wait()
                cps[i + 1] = gather(i + 1)
            cps[i].wait()
            scs[i] = scatter(i)
        if n_ch >= 2:
            scs[n_ch - 2].wait()
        scs[n_ch - 1].wait()

    return gk(table, idx)


def _gather_rows(table, idx):
    V, D = table.shape
    Dp = -(-D // 16) * 16
    if Dp != D:
        table = jnp.pad(table, ((0, 0), (0, Dp - D)))
    Bi = idx.shape[0]
    Bp = -(-Bi // 256) * 256
    idx_p = jnp.pad(idx, (0, Bp - Bi)) if Bp != Bi else idx
    rows = _sc_gather(table, idx_p.astype(jnp.int32))
    return rows[:Bi, :D]


def _mlp(x3, layers, pool, prep=None, extras=(), cprep=None):
    k, mp, cin = x3.shape
    cw = cprep if cprep is not None else cin
    cout = layers[-1][0].shape[1] if layers else cw
    gm = min(mp, 512)
    while gm > 8 and k * gm * max(cin, cw, cout) * 4 > 4 * 1024 * 1024:
        gm //= 2
    while mp % gm:
        gm //= 2
    ops = [x3]
    in_specs = [pl.BlockSpec((k, gm, cin), lambda i: (0, i, 0))]
    for e in extras:
        ops.append(e)
        ce = e.shape[1]
        in_specs.append(pl.BlockSpec((gm, ce), lambda i: (i, 0)))
    for (W, b, _r) in layers:
        ops.append(W)
        in_specs.append(pl.BlockSpec(W.shape, lambda i: (0, 0)))
        if b is not None:
            ops.append(b.reshape(1, -1))
            in_specs.append(pl.BlockSpec((1, b.size), lambda i: (0, 0)))
    ne = len(extras)

    def body(*refs):
        x_ref, o_ref = refs[0], refs[-1]
        e_vals = [r[...] for r in refs[1:1 + ne]]
        w_refs = refs[1 + ne:-1]

        def chain(x):
            wi = 0
            for (W, b, relu) in layers:
                x = lax.dot_general(x, w_refs[wi][...],
                                    (((1,), (0,)), ((), ())),
                                    preferred_element_type=jnp.float32)
                wi += 1
                if b is not None:
                    x = x + w_refs[wi][...]
                    wi += 1
                if relu:
                    x = jnp.maximum(x, 0.0)
            return x

        def make_x(j):
            xr = x_ref[j]
            return prep(xr, *e_vals) if prep is not None else xr

        if pool == 'max':
            def jstep(j, acc):
                return jnp.maximum(acc, chain(make_x(j)))
            o_ref[...] = lax.fori_loop(0, k, jstep,
                                       jnp.full((gm, cout), -_BIG, jnp.float32))
        elif pool == 'interp3':
            d_v, f1_v = e_vals
            dd = jnp.maximum(d_v, 1e-10)
            w = 1.0 / dd
            w = w / jnp.sum(w, axis=1, keepdims=True)

            def wj(j):
                return jnp.broadcast_to(w[:, j:j + 1], (gm, cin))

            xi = (x_ref[0] * wj(0) + x_ref[1] * wj(1)) + x_ref[2] * wj(2)
            o_ref[...] = chain(jnp.concatenate([xi, f1_v], axis=1))
        else:
            o_ref[...] = chain(make_x(0))

    return pl.pallas_call(
        body,
        grid=(mp // gm,),
        in_specs=in_specs,
        out_specs=pl.BlockSpec((gm, cout), lambda i: (i, 0)),
        out_shape=jax.ShapeDtypeStruct((mp, cout), jnp.float32),
    )(*ops)


def _offs(nb, n):
    return (jnp.arange(nb, dtype=jnp.int32) * n)[:, None, None]


def _grouped_rows(points, feats, idx):
    nb, n, _ = points.shape
    c = feats.shape[-1]
    k = idx.shape[-1]
    m = idx.shape[1]
    table = jnp.concatenate([points, feats], -1).reshape(nb * n, 3 + c)
    idx_f = jnp.transpose(idx + _offs(nb, n), (2, 0, 1)).reshape(-1)
    return _gather_rows(table, idx_f).reshape(k, nb * m, 3 + c)


def _sa(xyz, feat, npoint, k, Ws):
    nb, n, _ = xyz.shape
    if npoint < n:
        fidx = _fps_b(xyz, npoint)
        tab = xyz.reshape(nb * n, 3)
        gidx = (fidx + jnp.arange(nb, dtype=jnp.int32)[:, None] * n).reshape(-1)
        new_xyz = _gather_rows(tab, gidx).reshape(nb, npoint, 3)
    else:
        new_xyz = xyz
    idx, _ = _knn_b(new_xyz, xyz, k)
    rows = _grouped_rows(xyz, feat, idx)
    q = new_xyz.reshape(nb * npoint, 3)

    def prep(xr, qb):
        return jnp.concatenate([xr[:, :3] - qb, xr[:, 3:]], axis=1)

    out = _mlp(rows, [(W, None, True) for W in Ws], 'max',
               prep=prep, extras=(q,))
    return new_xyz, out.reshape(nb, npoint, -1)


def _flow_embedding(p1, p2, f1, f2, k, Ws):
    nb, m, _ = p1.shape
    idx, _ = _knn_b(p1, p2, k)
    rows = _grouped_rows(p2, f2, idx)
    q = p1.reshape(nb * m, 3)
    f1r = f1.reshape(nb * m, -1)
    c2 = f2.shape[-1]
    c1 = f1r.shape[-1]

    def prep(xr, qb, f1b):
        return jnp.concatenate([xr[:, 3:], f1b, xr[:, :3] - qb], axis=1)

    out = _mlp(rows, [(W, None, True) for W in Ws], 'max',
               prep=prep, extras=(q, f1r), cprep=c2 + c1 + 3)
    return out.reshape(nb, m, -1)


def _set_upconv(p1, p2, f1, f2, k, mlp_w, mlp2_w):
    nb, m, _ = p1.shape
    idx, _ = _knn_b(p1, p2, k)
    rows = _grouped_rows(p2, f2, idx)
    q = p1.reshape(nb * m, 3)

    def prep(xr, qb):
        return jnp.concatenate([xr[:, 3:], xr[:, :3] - qb], axis=1)

    pooled = _mlp(rows, [(W, None, True) for W in mlp_w], 'max',
                  prep=prep, extras=(q,))
    f1r = f1.reshape(nb * m, -1)

    def prep2(xr, f1b):
        return jnp.concatenate([xr, f1b], axis=1)

    out = _mlp(pooled[None], [(W, None, True) for W in mlp2_w], 'none',
               prep=prep2, extras=(f1r,),
               cprep=pooled.shape[-1] + f1r.shape[-1])
    return out.reshape(nb, m, -1)


def _feature_prop(p1, p2, f1, f2, Ws):
    nb, m, _ = p1.shape
    n = p2.shape[1]
    c = f2.shape[-1]
    idx, d = _knn_b(p1, p2, 3)
    idx_f = jnp.transpose(idx + _offs(nb, n), (2, 0, 1)).reshape(-1)
    rows = _gather_rows(f2.reshape(nb * n, c), idx_f).reshape(3, nb * m, c)
    f1r = f1.reshape(nb * m, -1)
    out = _mlp(rows, [(W, None, True) for W in Ws], 'interp3',
               extras=(d.reshape(nb * m, 3), f1r),
               cprep=c + f1r.shape[-1])
    return out.reshape(nb, m, -1)


def kernel(pc1, pc2, feature1, feature2, params):
    P = params
    x1 = pc1.transpose(0, 2, 1)
    x2 = pc2.transpose(0, 2, 1)
    ft1 = feature1.transpose(0, 2, 1)
    ft2 = feature2.transpose(0, 2, 1)

    xyz0 = jnp.concatenate([x1, x2], 0)
    feat0 = jnp.concatenate([ft1, ft2], 0)

    l0p, l0f = _sa(xyz0, feat0, 2048, 16, P['sa0'])
    l1p, l1f = _sa(l0p, l0f, 2048, 16, P['sa1'])
    l2p, l2f = _sa(l1p, l1f, 512, 16, P['sa2'])

    l2p1, l2p2 = l2p[:2], l2p[2:]
    l2f1, l2f2 = l2f[:2], l2f[2:]
    l1p1, l1f1 = l1p[:2], l1f[:2]

    l2f1n = _flow_embedding(l2p1, l2p2, l2f1, l2f2, 64, P['fe'])

    l3p1, l3f1 = _sa(l2p1, l2f1n, 128, 8, P['sa3'])
    l4p1, l4f1 = _sa(l3p1, l3f1, 32, 8, P['sa4'])

    l3fn = _set_upconv(l3p1, l4p1, l3f1, l4f1, 8, [], P['su1_mlp2'])
    l2fn = _set_upconv(l2p1, l3p1,
                       jnp.concatenate([l2f1, l2f1n], -1), l3fn, 8,
                       P['su2_mlp'], P['su2_mlp2'])
    l1fn = _set_upconv(l1p1, l2p1, l1f1, l2fn, 8, P['su3_mlp'], P['su3_mlp2'])

    l0fn = _feature_prop(x1, l1p1, ft1, l1fn, P['fp'])

    out = _mlp(l0fn.reshape(1, 2 * 8192, 256),
               [(P['conv1'], None, True),
                (P['conv2_w'], P['conv2_b'], False)], 'none')
    return out.reshape(2, 8192, 3)


# speedup vs baseline: 1.5588x; 1.1712x over previous
import functools

import jax
import jax.numpy as jnp
from jax import lax
from jax.experimental import pallas as pl
from jax.experimental.pallas import tpu as pltpu
from jax.experimental.pallas import tpu_sc as plsc

_BIG = float(3.0e38)


def _fps_b(xyz, npoint):
    nb, n, _ = xyz.shape
    cols = 128
    rows = max(1, -(-n // cols))
    rows8 = -(-rows // 8) * 8
    total = rows8 * cols
    pad = total - n
    if pad:
        xyz_p = jnp.concatenate(
            [xyz, jnp.broadcast_to(xyz[:, 0:1, :], (nb, pad, 3))], axis=1)
    else:
        xyz_p = xyz
    planes = xyz_p.transpose(0, 2, 1).reshape(nb, 3, rows8, cols)

    planes3 = jnp.transpose(xyz_p, (2, 0, 1)).reshape(3, nb, rows8, cols)

    def body(planes_ref, rows_ref, *rest):
        out_refs = rest[:nb]
        dists_ref = rest[nb]
        r_iota = lax.broadcasted_iota(jnp.int32, (nb, rows8, cols), 1)
        c_iota = lax.broadcasted_iota(jnp.int32, (nb, rows8, cols), 2)
        flat = r_iota * cols + c_iota
        dists_ref[...] = jnp.full((nb, rows8, cols), 1e10, jnp.float32)

        def step(j, fars):
            crows = [rows_ref[c, pl.ds(fars[c], 1), :] for c in range(nb)]
            for c in range(nb):
                out_refs[c][0, j] = fars[c]
            cxyz = jnp.concatenate(crows, axis=0)

            def cplane(a):
                return jnp.broadcast_to(cxyz[:, a][:, None, None],
                                        (nb, rows8, cols))

            dx = planes_ref[0] - cplane(0)
            dy = planes_ref[1] - cplane(1)
            dz = planes_ref[2] - cplane(2)
            d = dx * dx + dy * dy + dz * dz
            nd = jnp.minimum(dists_ref[...], d)
            dists_ref[...] = nd
            mx = jnp.max(nd, axis=(1, 2), keepdims=True)
            fidx = jnp.min(jnp.where(nd == mx, flat, total),
                           axis=(1, 2))
            nbi = lax.broadcasted_iota(jnp.int32, (nb,), 0)
            return tuple(
                jnp.min(jnp.where(nbi == c, fidx, total)).astype(jnp.int32)
                for c in range(nb))

        lax.fori_loop(0, npoint, step, tuple(jnp.int32(0) for _ in range(nb)))

    outs = pl.pallas_call(
        body,
        in_specs=[
            pl.BlockSpec(memory_space=pltpu.VMEM),
            pl.BlockSpec(memory_space=pltpu.VMEM),
        ],
        out_specs=[pl.BlockSpec(memory_space=pltpu.SMEM)] * nb,
        out_shape=[jax.ShapeDtypeStruct((1, npoint), jnp.int32)] * nb,
        scratch_shapes=[pltpu.VMEM((nb, rows8, cols), jnp.float32)],
    )(planes3, xyz_p)
    return jnp.concatenate(outs, axis=0)


def _knn_b(query, points, k):
    nb, m, _ = query.shape
    n = points.shape[1]
    bm = min(m, max(64, min(256, (8192 * 128) // n)))
    qp = jnp.pad(query, ((0, 0), (0, 0), (0, 5)))
    dt = jnp.pad(points.transpose(0, 2, 1), ((0, 0), (0, 5), (0, 0)))

    def body(q_ref, dt_ref, idx_ref, d_ref):
        q = q_ref[0]
        dtm = dt_ref[0]
        qs = jnp.sum(q * q, axis=1, keepdims=True)
        ps = jnp.sum(dtm * dtm, axis=0, keepdims=True)
        prod = lax.dot_general(q, dtm, (((1,), (0,)), ((), ())),
                               preferred_element_type=jnp.float32)
        cur = (-2.0 * prod + qs) + ps
        lane = lax.broadcasted_iota(jnp.int32, (bm, n), 1)
        idx_cols, d_cols = [], []
        for _ in range(k):
            dmin = jnp.min(cur, axis=1, keepdims=True)
            sel = cur == dmin
            ij = jnp.min(jnp.where(sel, lane, n), axis=1, keepdims=True)
            idx_cols.append(ij)
            d_cols.append(dmin)
            cur = jnp.where(lane == ij, _BIG, cur)
        idx_ref[0] = jnp.concatenate(idx_cols, axis=1)
        d_ref[0] = jnp.concatenate(d_cols, axis=1)

    idx, d = pl.pallas_call(
        body,
        grid=(nb, m // bm),
        in_specs=[
            pl.BlockSpec((1, bm, 8), lambda b, i: (b, i, 0)),
            pl.BlockSpec((1, 8, n), lambda b, i: (b, 0, 0)),
        ],
        out_specs=[
            pl.BlockSpec((1, bm, k), lambda b, i: (b, i, 0)),
            pl.BlockSpec((1, bm, k), lambda b, i: (b, i, 0)),
        ],
        out_shape=[
            jax.ShapeDtypeStruct((nb, m, k), jnp.int32),
            jax.ShapeDtypeStruct((nb, m, k), jnp.float32),
        ],
    )(qp, dt)
    return idx, d


def _sc_gather(table, idx):
    V, D = table.shape
    Bi = idx.shape[0]
    info = plsc.get_sparse_core_info()
    NC, NS = info.num_cores, info.num_subcores
    NW = NC * NS
    b_per_w = Bi // NW
    CH = min(b_per_w, 128)
    n_ch = b_per_w // CH
    mesh = plsc.VectorSubcoreMesh(core_axis_name="c", subcore_axis_name="s")

    @functools.partial(
        pl.kernel, mesh=mesh,
        compiler_params=pltpu.CompilerParams(use_tc_tiling_on_sc=False),
        out_type=jax.ShapeDtypeStruct((Bi, D), jnp.float32),
        scratch_types=[
            pltpu.VMEM((b_per_w,), jnp.int32),
            pltpu.VMEM((CH, D), jnp.float32),
            pltpu.VMEM((CH, D), jnp.float32),
            pltpu.SemaphoreType.DMA,
            pltpu.SemaphoreType.DMA,
            pltpu.SemaphoreType.DMA,
            pltpu.SemaphoreType.DMA,
        ],
    )
    def gk(table_hbm, idx_hbm, out_hbm, idx_v, rv0, rv1, gs0, gs1, ss0, ss1):
        wid = lax.axis_index("s") * NC + lax.axis_index("c")
        base = wid * b_per_w
        pltpu.sync_copy(idx_hbm.at[pl.ds(base, b_per_w)], idx_v)
        bufs, gsems, ssems = (rv0, rv1), (gs0, gs1), (ss0, ss1)

        def gather(i):
            return pltpu.async_copy(
                table_hbm.at[idx_v.at[pl.ds(i * CH, CH)]],
                bufs[i % 2], gsems[i % 2])

        def scatter(i):
            return pltpu.async_copy(
                bufs[i % 2], out_hbm.at[pl.ds(base + i * CH, CH)],
                ssems[i % 2])

        cps, scs = {}, {}
        cps[0] = gather(0)
        for i in range(n_ch):
            if i + 1 < n_ch:
                if i >= 1:
                    scs[i - 1].wait()
                cps[i + 1] = gather(i + 1)
            cps[i].wait()
            scs[i] = scatter(i)
        if n_ch >= 2:
            scs[n_ch - 2].wait()
        scs[n_ch - 1].wait()

    return gk(table, idx)


def _gather_rows(table, idx):
    V, D = table.shape
    Dp = -(-D // 16) * 16
    if Dp != D:
        table = jnp.pad(table, ((0, 0), (0, Dp - D)))
    Bi = idx.shape[0]
    Bp = -(-Bi // 256) * 256
    idx_p = jnp.pad(idx, (0, Bp - Bi)) if Bp != Bi else idx
    rows = _sc_gather(table, idx_p.astype(jnp.int32))
    return rows[:Bi, :D]


def _mlp(x3, layers, pool, prep=None, extras=(), cprep=None):
    k, mp, cin = x3.shape
    cw = cprep if cprep is not None else cin
    cout = layers[-1][0].shape[1] if layers else cw
    gm = min(mp, 512)
    while gm > 8 and k * gm * max(cin, cw, cout) * 4 > 4 * 1024 * 1024:
        gm //= 2
    while mp % gm:
        gm //= 2
    ops = [x3]
    in_specs = [pl.BlockSpec((k, gm, cin), lambda i: (0, i, 0))]
    for e in extras:
        ops.append(e)
        ce = e.shape[1]
        in_specs.append(pl.BlockSpec((gm, ce), lambda i: (i, 0)))
    for (W, b, _r) in layers:
        ops.append(W)
        in_specs.append(pl.BlockSpec(W.shape, lambda i: (0, 0)))
        if b is not None:
            ops.append(b.reshape(1, -1))
            in_specs.append(pl.BlockSpec((1, b.size), lambda i: (0, 0)))
    ne = len(extras)

    def body(*refs):
        x_ref, o_ref = refs[0], refs[-1]
        e_vals = [r[...] for r in refs[1:1 + ne]]
        w_refs = refs[1 + ne:-1]

        def chain(x):
            wi = 0
            for (W, b, relu) in layers:
                x = lax.dot_general(x, w_refs[wi][...],
                                    (((1,), (0,)), ((), ())),
                                    preferred_element_type=jnp.float32)
                wi += 1
                if b is not None:
                    x = x + w_refs[wi][...]
                    wi += 1
                if relu:
                    x = jnp.maximum(x, 0.0)
            return x

        def make_x(j):
            xr = x_ref[j]
            return prep(xr, *e_vals) if prep is not None else xr

        if pool == 'max':
            def jstep(j, acc):
                return jnp.maximum(acc, chain(make_x(j)))
            o_ref[...] = lax.fori_loop(0, k, jstep,
                                       jnp.full((gm, cout), -_BIG, jnp.float32))
        elif pool == 'interp3':
            d_v, f1_v = e_vals
            dd = jnp.maximum(d_v, 1e-10)
            w = 1.0 / dd
            w = w / jnp.sum(w, axis=1, keepdims=True)

            def wj(j):
                return jnp.broadcast_to(w[:, j:j + 1], (gm, cin))

            xi = (x_ref[0] * wj(0) + x_ref[1] * wj(1)) + x_ref[2] * wj(2)
            o_ref[...] = chain(jnp.concatenate([xi, f1_v], axis=1))
        else:
            o_ref[...] = chain(make_x(0))

    return pl.pallas_call(
        body,
        grid=(mp // gm,),
        in_specs=in_specs,
        out_specs=pl.BlockSpec((gm, cout), lambda i: (i, 0)),
        out_shape=jax.ShapeDtypeStruct((mp, cout), jnp.float32),
    )(*ops)


def _offs(nb, n):
    return (jnp.arange(nb, dtype=jnp.int32) * n)[:, None, None]


def _grouped_rows(points, feats, idx):
    nb, n, _ = points.shape
    c = feats.shape[-1]
    k = idx.shape[-1]
    m = idx.shape[1]
    table = jnp.concatenate([points, feats], -1).reshape(nb * n, 3 + c)
    idx_f = jnp.transpose(idx + _offs(nb, n), (2, 0, 1)).reshape(-1)
    return _gather_rows(table, idx_f).reshape(k, nb * m, 3 + c)


def _sa(xyz, feat, npoint, k, Ws):
    nb, n, _ = xyz.shape
    if npoint < n:
        fidx = _fps_b(xyz, npoint)
        tab = xyz.reshape(nb * n, 3)
        gidx = (fidx + jnp.arange(nb, dtype=jnp.int32)[:, None] * n).reshape(-1)
        new_xyz = _gather_rows(tab, gidx).reshape(nb, npoint, 3)
    else:
        new_xyz = xyz
    idx, _ = _knn_b(new_xyz, xyz, k)
    rows = _grouped_rows(xyz, feat, idx)
    q = new_xyz.reshape(nb * npoint, 3)

    def prep(xr, qb):
        return jnp.concatenate([xr[:, :3] - qb, xr[:, 3:]], axis=1)

    out = _mlp(rows, [(W, None, True) for W in Ws], 'max',
               prep=prep, extras=(q,))
    return new_xyz, out.reshape(nb, npoint, -1)


def _flow_embedding(p1, p2, f1, f2, k, Ws):
    nb, m, _ = p1.shape
    idx, _ = _knn_b(p1, p2, k)
    rows = _grouped_rows(p2, f2, idx)
    q = p1.reshape(nb * m, 3)
    f1r = f1.reshape(nb * m, -1)
    c2 = f2.shape[-1]
    c1 = f1r.shape[-1]

    def prep(xr, qb, f1b):
        return jnp.concatenate([xr[:, 3:], f1b, xr[:, :3] - qb], axis=1)

    out = _mlp(rows, [(W, None, True) for W in Ws], 'max',
               prep=prep, extras=(q, f1r), cprep=c2 + c1 + 3)
    return out.reshape(nb, m, -1)


def _set_upconv(p1, p2, f1, f2, k, mlp_w, mlp2_w):
    nb, m, _ = p1.shape
    idx, _ = _knn_b(p1, p2, k)
    rows = _grouped_rows(p2, f2, idx)
    q = p1.reshape(nb * m, 3)

    def prep(xr, qb):
        return jnp.concatenate([xr[:, 3:], xr[:, :3] - qb], axis=1)

    pooled = _mlp(rows, [(W, None, True) for W in mlp_w], 'max',
                  prep=prep, extras=(q,))
    f1r = f1.reshape(nb * m, -1)

    def prep2(xr, f1b):
        return jnp.concatenate([xr, f1b], axis=1)

    out = _mlp(pooled[None], [(W, None, True) for W in mlp2_w], 'none',
               prep=prep2, extras=(f1r,),
               cprep=pooled.shape[-1] + f1r.shape[-1])
    return out.reshape(nb, m, -1)


def _feature_prop(p1, p2, f1, f2, Ws):
    nb, m, _ = p1.shape
    n = p2.shape[1]
    c = f2.shape[-1]
    idx, d = _knn_b(p1, p2, 3)
    idx_f = jnp.transpose(idx + _offs(nb, n), (2, 0, 1)).reshape(-1)
    rows = _gather_rows(f2.reshape(nb * n, c), idx_f).reshape(3, nb * m, c)
    f1r = f1.reshape(nb * m, -1)
    out = _mlp(rows, [(W, None, True) for W in Ws], 'interp3',
               extras=(d.reshape(nb * m, 3), f1r),
               cprep=c + f1r.shape[-1])
    return out.reshape(nb, m, -1)


def kernel(pc1, pc2, feature1, feature2, params):
    P = params
    x1 = pc1.transpose(0, 2, 1)
    x2 = pc2.transpose(0, 2, 1)
    ft1 = feature1.transpose(0, 2, 1)
    ft2 = feature2.transpose(0, 2, 1)

    xyz0 = jnp.concatenate([x1, x2], 0)
    feat0 = jnp.concatenate([ft1, ft2], 0)

    l0p, l0f = _sa(xyz0, feat0, 2048, 16, P['sa0'])
    l1p, l1f = _sa(l0p, l0f, 2048, 16, P['sa1'])
    l2p, l2f = _sa(l1p, l1f, 512, 16, P['sa2'])

    l2p1, l2p2 = l2p[:2], l2p[2:]
    l2f1, l2f2 = l2f[:2], l2f[2:]
    l1p1, l1f1 = l1p[:2], l1f[:2]

    l2f1n = _flow_embedding(l2p1, l2p2, l2f1, l2f2, 64, P['fe'])

    l3p1, l3f1 = _sa(l2p1, l2f1n, 128, 8, P['sa3'])
    l4p1, l4f1 = _sa(l3p1, l3f1, 32, 8, P['sa4'])

    l3fn = _set_upconv(l3p1, l4p1, l3f1, l4f1, 8, [], P['su1_mlp2'])
    l2fn = _set_upconv(l2p1, l3p1,
                       jnp.concatenate([l2f1, l2f1n], -1), l3fn, 8,
                       P['su2_mlp'], P['su2_mlp2'])
    l1fn = _set_upconv(l1p1, l2p1, l1f1, l2fn, 8, P['su3_mlp'], P['su3_mlp2'])

    l0fn = _feature_prop(x1, l1p1, ft1, l1fn, P['fp'])

    out = _mlp(l0fn.reshape(1, 2 * 8192, 256),
               [(P['conv1'], None, True),
                (P['conv2_w'], P['conv2_b'], False)], 'none')
    return out.reshape(2, 8192, 3)


# speedup vs baseline: 1.6104x; 1.0331x over previous
import functools

import jax
import jax.numpy as jnp
from jax import lax
from jax.experimental import pallas as pl
from jax.experimental.pallas import tpu as pltpu
from jax.experimental.pallas import tpu_sc as plsc

_BIG = float(3.0e38)


def _fps_b(xyz, npoint):
    nb, n, _ = xyz.shape
    cols = 128
    rows = max(1, -(-n // cols))
    rows8 = -(-rows // 8) * 8
    total = rows8 * cols
    pad = total - n
    if pad:
        xyz_p = jnp.concatenate(
            [xyz, jnp.broadcast_to(xyz[:, 0:1, :], (nb, pad, 3))], axis=1)
    else:
        xyz_p = xyz
    planes = xyz_p.transpose(0, 2, 1).reshape(nb, 3, rows8, cols)

    planes3 = jnp.transpose(xyz_p, (2, 0, 1)).reshape(3, nb, rows8, cols)

    def body(planes_ref, rows_ref, *rest):
        out_refs = rest[:nb]
        dists_ref = rest[nb]
        r_iota = lax.broadcasted_iota(jnp.int32, (nb, rows8, cols), 1)
        c_iota = lax.broadcasted_iota(jnp.int32, (nb, rows8, cols), 2)
        flat = r_iota * cols + c_iota
        dists_ref[...] = jnp.full((nb, rows8, cols), 1e10, jnp.float32)

        def step(j, fars):
            crows = [rows_ref[c, pl.ds(fars[c], 1), :] for c in range(nb)]
            for c in range(nb):
                out_refs[c][0, j] = fars[c]
            cxyz = jnp.concatenate(crows, axis=0)

            def cplane(a):
                return jnp.broadcast_to(cxyz[:, a][:, None, None],
                                        (nb, rows8, cols))

            dx = planes_ref[0] - cplane(0)
            dy = planes_ref[1] - cplane(1)
            dz = planes_ref[2] - cplane(2)
            d = dx * dx + dy * dy + dz * dz
            nd = jnp.minimum(dists_ref[...], d)
            dists_ref[...] = nd
            mx = jnp.max(nd, axis=(1, 2), keepdims=True)
            fidx = jnp.min(jnp.where(nd == mx, flat, total),
                           axis=(1, 2))
            nbi = lax.broadcasted_iota(jnp.int32, (nb,), 0)
            return tuple(
                jnp.min(jnp.where(nbi == c, fidx, total)).astype(jnp.int32)
                for c in range(nb))

        lax.fori_loop(0, npoint, step, tuple(jnp.int32(0) for _ in range(nb)))

    outs = pl.pallas_call(
        body,
        in_specs=[
            pl.BlockSpec(memory_space=pltpu.VMEM),
            pl.BlockSpec(memory_space=pltpu.VMEM),
        ],
        out_specs=[pl.BlockSpec(memory_space=pltpu.SMEM)] * nb,
        out_shape=[jax.ShapeDtypeStruct((1, npoint), jnp.int32)] * nb,
        scratch_shapes=[pltpu.VMEM((nb, rows8, cols), jnp.float32)],
    )(planes3, xyz_p)
    return jnp.concatenate(outs, axis=0)


def _knn_b(query, points, k):
    nb, m, _ = query.shape
    n = points.shape[1]
    bm = min(m, max(64, min(256, (8192 * 128) // n)))
    qp = jnp.pad(query, ((0, 0), (0, 0), (0, 5)))
    dt = jnp.pad(points.transpose(0, 2, 1), ((0, 0), (0, 5), (0, 0)))

    def body(q_ref, dt_ref, idx_ref, d_ref):
        q = q_ref[0]
        dtm = dt_ref[0]
        qs = jnp.sum(q * q, axis=1, keepdims=True)
        ps = jnp.sum(dtm * dtm, axis=0, keepdims=True)
        prod = lax.dot_general(q, dtm, (((1,), (0,)), ((), ())),
                               preferred_element_type=jnp.float32)
        cur = (-2.0 * prod + qs) + ps
        lane = lax.broadcasted_iota(jnp.int32, (bm, n), 1)
        idx_cols, d_cols = [], []
        for _ in range(k):
            dmin = jnp.min(cur, axis=1, keepdims=True)
            sel = cur == dmin
            ij = jnp.min(jnp.where(sel, lane, n), axis=1, keepdims=True)
            idx_cols.append(ij)
            d_cols.append(dmin)
            cur = jnp.where(lane == ij, _BIG, cur)
        idx_ref[0] = jnp.concatenate(idx_cols, axis=1)
        d_ref[0] = jnp.concatenate(d_cols, axis=1)

    idx, d = pl.pallas_call(
        body,
        grid=(nb, m // bm),
        in_specs=[
            pl.BlockSpec((1, bm, 8), lambda b, i: (b, i, 0)),
            pl.BlockSpec((1, 8, n), lambda b, i: (b, 0, 0)),
        ],
        out_specs=[
            pl.BlockSpec((1, bm, k), lambda b, i: (b, i, 0)),
            pl.BlockSpec((1, bm, k), lambda b, i: (b, i, 0)),
        ],
        out_shape=[
            jax.ShapeDtypeStruct((nb, m, k), jnp.int32),
            jax.ShapeDtypeStruct((nb, m, k), jnp.float32),
        ],
    )(qp, dt)
    return idx, d


def _sc_gather(table, idx):
    V, D = table.shape
    Bi = idx.shape[0]
    info = plsc.get_sparse_core_info()
    NC, NS = info.num_cores, info.num_subcores
    NW = NC * NS
    b_per_w = Bi // NW
    CH = min(b_per_w, 128)
    n_ch = b_per_w // CH
    mesh = plsc.VectorSubcoreMesh(core_axis_name="c", subcore_axis_name="s")

    @functools.partial(
        pl.kernel, mesh=mesh,
        compiler_params=pltpu.CompilerParams(use_tc_tiling_on_sc=False),
        out_type=jax.ShapeDtypeStruct((Bi, D), jnp.float32),
        scratch_types=[
            pltpu.VMEM((b_per_w,), jnp.int32),
            pltpu.VMEM((CH, D), jnp.float32),
            pltpu.VMEM((CH, D), jnp.float32),
            pltpu.SemaphoreType.DMA,
            pltpu.SemaphoreType.DMA,
            pltpu.SemaphoreType.DMA,
            pltpu.SemaphoreType.DMA,
        ],
    )
    def gk(table_hbm, idx_hbm, out_hbm, idx_v, rv0, rv1, gs0, gs1, ss0, ss1):
        wid = lax.axis_index("s") * NC + lax.axis_index("c")
        base = wid * b_per_w
        pltpu.sync_copy(idx_hbm.at[pl.ds(base, b_per_w)], idx_v)
        bufs, gsems, ssems = (rv0, rv1), (gs0, gs1), (ss0, ss1)

        def gather(i):
            return pltpu.async_copy(
                table_hbm.at[idx_v.at[pl.ds(i * CH, CH)]],
                bufs[i % 2], gsems[i % 2])

        def scatter(i):
            return pltpu.async_copy(
                bufs[i % 2], out_hbm.at[pl.ds(base + i * CH, CH)],
                ssems[i % 2])

        cps, scs = {}, {}
        cps[0] = gather(0)
        for i in range(n_ch):
            if i + 1 < n_ch:
                if i >= 1:
                    scs[i - 1].wait()
                cps[i + 1] = gather(i + 1)
            cps[i].wait()
            scs[i] = scatter(i)
        if n_ch >= 2:
            scs[n_ch - 2].wait()
        scs[n_ch - 1].wait()

    return gk(table, idx)


def _gather_rows(table, idx):
    V, D = table.shape
    Dp = -(-D // 16) * 16
    if Dp != D:
        table = jnp.pad(table, ((0, 0), (0, Dp - D)))
    Bi = idx.shape[0]
    Bp = -(-Bi // 256) * 256
    idx_p = jnp.pad(idx, (0, Bp - Bi)) if Bp != Bi else idx
    rows = _sc_gather(table, idx_p.astype(jnp.int32))
    return rows[:Bi, :D]


def _mlp(x3, layers, pool, prep=None, extras=(), cprep=None):
    k, mp, cin = x3.shape
    cw = cprep if cprep is not None else cin
    cout = layers[-1][0].shape[1] if layers else cw
    gm = min(mp, 512)
    while gm > 8 and k * gm * max(cin, cw, cout) * 4 > 4 * 1024 * 1024:
        gm //= 2
    while mp % gm:
        gm //= 2
    ops = [x3]
    in_specs = [pl.BlockSpec((k, gm, cin), lambda i: (0, i, 0))]
    for e in extras:
        ops.append(e)
        ce = e.shape[1]
        in_specs.append(pl.BlockSpec((gm, ce), lambda i: (i, 0)))
    for (W, b, _r) in layers:
        ops.append(W)
        in_specs.append(pl.BlockSpec(W.shape, lambda i: (0, 0)))
        if b is not None:
            ops.append(b.reshape(1, -1))
            in_specs.append(pl.BlockSpec((1, b.size), lambda i: (0, 0)))
    ne = len(extras)

    def body(*refs):
        x_ref, o_ref = refs[0], refs[-1]
        e_vals = [r[...] for r in refs[1:1 + ne]]
        w_refs = refs[1 + ne:-1]

        def chain(x):
            wi = 0
            for (W, b, relu) in layers:
                x = lax.dot_general(x, w_refs[wi][...],
                                    (((1,), (0,)), ((), ())),
                                    preferred_element_type=jnp.float32)
                wi += 1
                if b is not None:
                    x = x + w_refs[wi][...]
                    wi += 1
                if relu:
                    x = jnp.maximum(x, 0.0)
            return x

        def make_x(j):
            xr = x_ref[j]
            return prep(xr, *e_vals) if prep is not None else xr

        if pool == 'max':
            def jstep(j, acc):
                return jnp.maximum(acc, chain(make_x(j)))
            if k <= 16:
                acc = chain(make_x(0))
                for j in range(1, k):
                    acc = jstep(j, acc)
                o_ref[...] = acc
            else:
                o_ref[...] = lax.fori_loop(
                    0, k, jstep, jnp.full((gm, cout), -_BIG, jnp.float32))
        elif pool == 'interp3':
            d_v, f1_v = e_vals
            dd = jnp.maximum(d_v, 1e-10)
            w = 1.0 / dd
            w = w / jnp.sum(w, axis=1, keepdims=True)

            def wj(j):
                return jnp.broadcast_to(w[:, j:j + 1], (gm, cin))

            xi = (x_ref[0] * wj(0) + x_ref[1] * wj(1)) + x_ref[2] * wj(2)
            o_ref[...] = chain(jnp.concatenate([xi, f1_v], axis=1))
        else:
            o_ref[...] = chain(make_x(0))

    return pl.pallas_call(
        body,
        grid=(mp // gm,),
        in_specs=in_specs,
        out_specs=pl.BlockSpec((gm, cout), lambda i: (i, 0)),
        out_shape=jax.ShapeDtypeStruct((mp, cout), jnp.float32),
    )(*ops)


def _offs(nb, n):
    return (jnp.arange(nb, dtype=jnp.int32) * n)[:, None, None]


def _grouped_rows(points, feats, idx):
    nb, n, _ = points.shape
    c = feats.shape[-1]
    k = idx.shape[-1]
    m = idx.shape[1]
    table = jnp.concatenate([points, feats], -1).reshape(nb * n, 3 + c)
    idx_f = jnp.transpose(idx + _offs(nb, n), (2, 0, 1)).reshape(-1)
    return _gather_rows(table, idx_f).reshape(k, nb * m, 3 + c)


def _sa(xyz, feat, npoint, k, Ws):
    nb, n, _ = xyz.shape
    if npoint < n:
        fidx = _fps_b(xyz, npoint)
        tab = xyz.reshape(nb * n, 3)
        gidx = (fidx + jnp.arange(nb, dtype=jnp.int32)[:, None] * n).reshape(-1)
        new_xyz = _gather_rows(tab, gidx).reshape(nb, npoint, 3)
    else:
        new_xyz = xyz
    idx, _ = _knn_b(new_xyz, xyz, k)
    rows = _grouped_rows(xyz, feat, idx)
    q = new_xyz.reshape(nb * npoint, 3)

    def prep(xr, qb):
        return jnp.concatenate([xr[:, :3] - qb, xr[:, 3:]], axis=1)

    out = _mlp(rows, [(W, None, True) for W in Ws], 'max',
               prep=prep, extras=(q,))
    return new_xyz, out.reshape(nb, npoint, -1)


def _flow_embedding(p1, p2, f1, f2, k, Ws):
    nb, m, _ = p1.shape
    idx, _ = _knn_b(p1, p2, k)
    rows = _grouped_rows(p2, f2, idx)
    q = p1.reshape(nb * m, 3)
    f1r = f1.reshape(nb * m, -1)
    c2 = f2.shape[-1]
    c1 = f1r.shape[-1]

    def prep(xr, qb, f1b):
        return jnp.concatenate([xr[:, 3:], f1b, xr[:, :3] - qb], axis=1)

    out = _mlp(rows, [(W, None, True) for W in Ws], 'max',
               prep=prep, extras=(q, f1r), cprep=c2 + c1 + 3)
    return out.reshape(nb, m, -1)


def _set_upconv(p1, p2, f1, f2, k, mlp_w, mlp2_w):
    nb, m, _ = p1.shape
    idx, _ = _knn_b(p1, p2, k)
    rows = _grouped_rows(p2, f2, idx)
    q = p1.reshape(nb * m, 3)

    def prep(xr, qb):
        return jnp.concatenate([xr[:, 3:], xr[:, :3] - qb], axis=1)

    pooled = _mlp(rows, [(W, None, True) for W in mlp_w], 'max',
                  prep=prep, extras=(q,))
    f1r = f1.reshape(nb * m, -1)

    def prep2(xr, f1b):
        return jnp.concatenate([xr, f1b], axis=1)

    out = _mlp(pooled[None], [(W, None, True) for W in mlp2_w], 'none',
               prep=prep2, extras=(f1r,),
               cprep=pooled.shape[-1] + f1r.shape[-1])
    return out.reshape(nb, m, -1)


def _feature_prop(p1, p2, f1, f2, Ws):
    nb, m, _ = p1.shape
    n = p2.shape[1]
    c = f2.shape[-1]
    idx, d = _knn_b(p1, p2, 3)
    idx_f = jnp.transpose(idx + _offs(nb, n), (2, 0, 1)).reshape(-1)
    rows = _gather_rows(f2.reshape(nb * n, c), idx_f).reshape(3, nb * m, c)
    f1r = f1.reshape(nb * m, -1)
    out = _mlp(rows, [(W, None, True) for W in Ws], 'interp3',
               extras=(d.reshape(nb * m, 3), f1r),
               cprep=c + f1r.shape[-1])
    return out.reshape(nb, m, -1)


def kernel(pc1, pc2, feature1, feature2, params):
    P = params
    x1 = pc1.transpose(0, 2, 1)
    x2 = pc2.transpose(0, 2, 1)
    ft1 = feature1.transpose(0, 2, 1)
    ft2 = feature2.transpose(0, 2, 1)

    xyz0 = jnp.concatenate([x1, x2], 0)
    feat0 = jnp.concatenate([ft1, ft2], 0)

    l0p, l0f = _sa(xyz0, feat0, 2048, 16, P['sa0'])
    l1p, l1f = _sa(l0p, l0f, 2048, 16, P['sa1'])
    l2p, l2f = _sa(l1p, l1f, 512, 16, P['sa2'])

    l2p1, l2p2 = l2p[:2], l2p[2:]
    l2f1, l2f2 = l2f[:2], l2f[2:]
    l1p1, l1f1 = l1p[:2], l1f[:2]

    l2f1n = _flow_embedding(l2p1, l2p2, l2f1, l2f2, 64, P['fe'])

    l3p1, l3f1 = _sa(l2p1, l2f1n, 128, 8, P['sa3'])
    l4p1, l4f1 = _sa(l3p1, l3f1, 32, 8, P['sa4'])

    l3fn = _set_upconv(l3p1, l4p1, l3f1, l4f1, 8, [], P['su1_mlp2'])
    l2fn = _set_upconv(l2p1, l3p1,
                       jnp.concatenate([l2f1, l2f1n], -1), l3fn, 8,
                       P['su2_mlp'], P['su2_mlp2'])
    l1fn = _set_upconv(l1p1, l2p1, l1f1, l2fn, 8, P['su3_mlp'], P['su3_mlp2'])

    l0fn = _feature_prop(x1, l1p1, ft1, l1fn, P['fp'])

    out = _mlp(l0fn.reshape(1, 2 * 8192, 256),
               [(P['conv1'], None, True),
                (P['conv2_w'], P['conv2_b'], False)], 'none')
    return out.reshape(2, 8192, 3)


# speedup vs baseline: 1.7221x; 1.0694x over previous
import functools

import jax
import jax.numpy as jnp
from jax import lax
from jax.experimental import pallas as pl
from jax.experimental.pallas import tpu as pltpu
from jax.experimental.pallas import tpu_sc as plsc

_BIG = float(3.0e38)


def _fps_b(xyz, npoint):
    nb, n, _ = xyz.shape
    cols = 128
    rows = max(1, -(-n // cols))
    rows8 = -(-rows // 8) * 8
    total = rows8 * cols
    pad = total - n
    if pad:
        xyz_p = jnp.concatenate(
            [xyz, jnp.broadcast_to(xyz[:, 0:1, :], (nb, pad, 3))], axis=1)
    else:
        xyz_p = xyz
    planes = xyz_p.transpose(0, 2, 1).reshape(nb, 3, rows8, cols)

    planes3 = jnp.transpose(xyz_p, (2, 0, 1)).reshape(3, nb, rows8, cols)

    def body(planes_ref, rows_ref, *rest):
        out_refs = rest[:nb]
        dists_ref = rest[nb]
        r_iota = lax.broadcasted_iota(jnp.int32, (nb, rows8, cols), 1)
        c_iota = lax.broadcasted_iota(jnp.int32, (nb, rows8, cols), 2)
        flat = r_iota * cols + c_iota
        dists_ref[...] = jnp.full((nb, rows8, cols), 1e10, jnp.float32)

        def step(j, fars):
            crows = [rows_ref[c, pl.ds(fars[c], 1), :] for c in range(nb)]
            for c in range(nb):
                out_refs[c][0, j] = fars[c]
            cxyz = jnp.concatenate(crows, axis=0)

            def cplane(a):
                return jnp.broadcast_to(cxyz[:, a][:, None, None],
                                        (nb, rows8, cols))

            dx = planes_ref[0] - cplane(0)
            dy = planes_ref[1] - cplane(1)
            dz = planes_ref[2] - cplane(2)
            d = dx * dx + dy * dy + dz * dz
            nd = jnp.minimum(dists_ref[...], d)
            dists_ref[...] = nd
            mx = jnp.max(nd, axis=(1, 2), keepdims=True)
            fidx = jnp.min(jnp.where(nd == mx, flat, total),
                           axis=(1, 2))
            nbi = lax.broadcasted_iota(jnp.int32, (nb,), 0)
            return tuple(
                jnp.min(jnp.where(nbi == c, fidx, total)).astype(jnp.int32)
                for c in range(nb))

        lax.fori_loop(0, npoint, step, tuple(jnp.int32(0) for _ in range(nb)))

    outs = pl.pallas_call(
        body,
        in_specs=[
            pl.BlockSpec(memory_space=pltpu.VMEM),
            pl.BlockSpec(memory_space=pltpu.VMEM),
        ],
        out_specs=[pl.BlockSpec(memory_space=pltpu.SMEM)] * nb,
        out_shape=[jax.ShapeDtypeStruct((1, npoint), jnp.int32)] * nb,
        scratch_shapes=[pltpu.VMEM((nb, rows8, cols), jnp.float32)],
    )(planes3, xyz_p)
    return jnp.concatenate(outs, axis=0)


def _knn_b(query, points, k):
    nb, m, _ = query.shape
    n = points.shape[1]
    bm = min(m, max(64, min(256, (8192 * 128) // n)))
    qp = jnp.pad(query, ((0, 0), (0, 0), (0, 5)))
    dt = jnp.pad(points.transpose(0, 2, 1), ((0, 0), (0, 5), (0, 0)))

    def body(q_ref, dt_ref, idx_ref, d_ref):
        q = q_ref[0]
        dtm = dt_ref[0]
        qs = jnp.sum(q * q, axis=1, keepdims=True)
        ps = jnp.sum(dtm * dtm, axis=0, keepdims=True)
        prod = lax.dot_general(q, dtm, (((1,), (0,)), ((), ())),
                               preferred_element_type=jnp.float32)
        cur = (-2.0 * prod + qs) + ps
        lane = lax.broadcasted_iota(jnp.int32, (bm, n), 1)
        idx_cols, d_cols = [], []
        for _ in range(k):
            dmin = jnp.min(cur, axis=1, keepdims=True)
            sel = cur == dmin
            ij = jnp.min(jnp.where(sel, lane, n), axis=1, keepdims=True)
            idx_cols.append(ij)
            d_cols.append(dmin)
            cur = jnp.where(lane == ij, _BIG, cur)
        idx_ref[0] = jnp.concatenate(idx_cols, axis=1)
        d_ref[0] = jnp.concatenate(d_cols, axis=1)

    idx, d = pl.pallas_call(
        body,
        grid=(nb, m // bm),
        in_specs=[
            pl.BlockSpec((1, bm, 8), lambda b, i: (b, i, 0)),
            pl.BlockSpec((1, 8, n), lambda b, i: (b, 0, 0)),
        ],
        out_specs=[
            pl.BlockSpec((1, bm, k), lambda b, i: (b, i, 0)),
            pl.BlockSpec((1, bm, k), lambda b, i: (b, i, 0)),
        ],
        out_shape=[
            jax.ShapeDtypeStruct((nb, m, k), jnp.int32),
            jax.ShapeDtypeStruct((nb, m, k), jnp.float32),
        ],
    )(qp, dt)
    return idx, d


def _sc_gather(table, idx):
    V, D = table.shape
    Bi = idx.shape[0]
    info = plsc.get_sparse_core_info()
    NC, NS = info.num_cores, info.num_subcores
    NW = NC * NS
    b_per_w = Bi // NW
    CH = min(b_per_w, 128)
    n_ch = b_per_w // CH
    mesh = plsc.VectorSubcoreMesh(core_axis_name="c", subcore_axis_name="s")

    @functools.partial(
        pl.kernel, mesh=mesh,
        compiler_params=pltpu.CompilerParams(use_tc_tiling_on_sc=False),
        out_type=jax.ShapeDtypeStruct((Bi, D), jnp.float32),
        scratch_types=[
            pltpu.VMEM((b_per_w,), jnp.int32),
            pltpu.VMEM((CH, D), jnp.float32),
            pltpu.VMEM((CH, D), jnp.float32),
            pltpu.SemaphoreType.DMA,
            pltpu.SemaphoreType.DMA,
            pltpu.SemaphoreType.DMA,
            pltpu.SemaphoreType.DMA,
        ],
    )
    def gk(table_hbm, idx_hbm, out_hbm, idx_v, rv0, rv1, gs0, gs1, ss0, ss1):
        wid = lax.axis_index("s") * NC + lax.axis_index("c")
        base = wid * b_per_w
        pltpu.sync_copy(idx_hbm.at[pl.ds(base, b_per_w)], idx_v)
        bufs, gsems, ssems = (rv0, rv1), (gs0, gs1), (ss0, ss1)

        def gather(i):
            return pltpu.async_copy(
                table_hbm.at[idx_v.at[pl.ds(i * CH, CH)]],
                bufs[i % 2], gsems[i % 2])

        def scatter(i):
            return pltpu.async_copy(
                bufs[i % 2], out_hbm.at[pl.ds(base + i * CH, CH)],
                ssems[i % 2])

        cps, scs = {}, {}
        cps[0] = gather(0)
        for i in range(n_ch):
            if i + 1 < n_ch:
                if i >= 1:
                    scs[i - 1].wait()
                cps[i + 1] = gather(i + 1)
            cps[i].wait()
            scs[i] = scatter(i)
        if n_ch >= 2:
            scs[n_ch - 2].wait()
        scs[n_ch - 1].wait()

    return gk(table, idx)


def _gather_rows(table, idx):
    V, D = table.shape
    Dp = -(-D // 16) * 16
    if Dp != D:
        table = jnp.pad(table, ((0, 0), (0, Dp - D)))
    Bi = idx.shape[0]
    Bp = -(-Bi // 256) * 256
    idx_p = jnp.pad(idx, (0, Bp - Bi)) if Bp != Bi else idx
    rows = _sc_gather(table, idx_p.astype(jnp.int32))
    return rows[:Bi, :D]


def _mlp(x3, layers, pool, prep=None, extras=(), cprep=None):
    k, mp, cin = x3.shape
    cw = cprep if cprep is not None else cin
    cout = layers[-1][0].shape[1] if layers else cw
    gm = min(mp, 512)
    while gm > 8 and k * gm * max(cin, cw, cout) * 4 > 4 * 1024 * 1024:
        gm //= 2
    while mp % gm:
        gm //= 2
    ops = [x3]
    in_specs = [pl.BlockSpec((k, gm, cin), lambda i: (0, i, 0))]
    for e in extras:
        ops.append(e)
        ce = e.shape[1]
        in_specs.append(pl.BlockSpec((gm, ce), lambda i: (i, 0)))
    for (W, b, _r) in layers:
        ops.append(W)
        in_specs.append(pl.BlockSpec(W.shape, lambda i: (0, 0)))
        if b is not None:
            ops.append(b.reshape(1, -1))
            in_specs.append(pl.BlockSpec((1, b.size), lambda i: (0, 0)))
    ne = len(extras)

    def body(*refs):
        x_ref, o_ref = refs[0], refs[-1]
        e_vals = [r[...] for r in refs[1:1 + ne]]
        w_refs = refs[1 + ne:-1]

        def chain(x):
            wi = 0
            for (W, b, relu) in layers:
                x = lax.dot_general(x, w_refs[wi][...],
                                    (((1,), (0,)), ((), ())),
                                    preferred_element_type=jnp.float32)
                wi += 1
                if b is not None:
                    x = x + w_refs[wi][...]
                    wi += 1
                if relu:
                    x = jnp.maximum(x, 0.0)
            return x

        def make_x(j):
            xr = x_ref[j]
            return prep(xr, *e_vals) if prep is not None else xr

        if pool == 'max':
            def jstep(j, acc):
                return jnp.maximum(acc, chain(make_x(j)))
            if k <= 16:
                acc = chain(make_x(0))
                for j in range(1, k):
                    acc = jstep(j, acc)
                o_ref[...] = acc
            else:
                o_ref[...] = lax.fori_loop(
                    0, k, jstep, jnp.full((gm, cout), -_BIG, jnp.float32),
                    unroll=8)
        elif pool == 'interp3':
            d_v, f1_v = e_vals
            dd = jnp.maximum(d_v, 1e-10)
            w = 1.0 / dd
            w = w / jnp.sum(w, axis=1, keepdims=True)

            def wj(j):
                return jnp.broadcast_to(w[:, j:j + 1], (gm, cin))

            xi = (x_ref[0] * wj(0) + x_ref[1] * wj(1)) + x_ref[2] * wj(2)
            o_ref[...] = chain(jnp.concatenate([xi, f1_v], axis=1))
        else:
            o_ref[...] = chain(make_x(0))

    return pl.pallas_call(
        body,
        grid=(mp // gm,),
        in_specs=in_specs,
        out_specs=pl.BlockSpec((gm, cout), lambda i: (i, 0)),
        out_shape=jax.ShapeDtypeStruct((mp, cout), jnp.float32),
    )(*ops)


def _offs(nb, n):
    return (jnp.arange(nb, dtype=jnp.int32) * n)[:, None, None]


def _grouped_rows(points, feats, idx):
    nb, n, _ = points.shape
    c = feats.shape[-1]
    k = idx.shape[-1]
    m = idx.shape[1]
    table = jnp.concatenate([points, feats], -1).reshape(nb * n, 3 + c)
    idx_f = jnp.transpose(idx + _offs(nb, n), (2, 0, 1)).reshape(-1)
    return _gather_rows(table, idx_f).reshape(k, nb * m, 3 + c)


def _sa(xyz, feat, npoint, k, Ws):
    nb, n, _ = xyz.shape
    if npoint < n:
        fidx = _fps_b(xyz, npoint)
        tab = xyz.reshape(nb * n, 3)
        gidx = (fidx + jnp.arange(nb, dtype=jnp.int32)[:, None] * n).reshape(-1)
        new_xyz = _gather_rows(tab, gidx).reshape(nb, npoint, 3)
    else:
        new_xyz = xyz
    idx, _ = _knn_b(new_xyz, xyz, k)
    rows = _grouped_rows(xyz, feat, idx)
    q = new_xyz.reshape(nb * npoint, 3)

    def prep(xr, qb):
        return jnp.concatenate([xr[:, :3] - qb, xr[:, 3:]], axis=1)

    out = _mlp(rows, [(W, None, True) for W in Ws], 'max',
               prep=prep, extras=(q,))
    return new_xyz, out.reshape(nb, npoint, -1)


def _flow_embedding(p1, p2, f1, f2, k, Ws):
    nb, m, _ = p1.shape
    idx, _ = _knn_b(p1, p2, k)
    rows = _grouped_rows(p2, f2, idx)
    q = p1.reshape(nb * m, 3)
    f1r = f1.reshape(nb * m, -1)
    c2 = f2.shape[-1]
    c1 = f1r.shape[-1]

    def prep(xr, qb, f1b):
        return jnp.concatenate([xr[:, 3:], f1b, xr[:, :3] - qb], axis=1)

    out = _mlp(rows, [(W, None, True) for W in Ws], 'max',
               prep=prep, extras=(q, f1r), cprep=c2 + c1 + 3)
    return out.reshape(nb, m, -1)


def _set_upconv(p1, p2, f1, f2, k, mlp_w, mlp2_w):
    nb, m, _ = p1.shape
    idx, _ = _knn_b(p1, p2, k)
    rows = _grouped_rows(p2, f2, idx)
    q = p1.reshape(nb * m, 3)

    def prep(xr, qb):
        return jnp.concatenate([xr[:, 3:], xr[:, :3] - qb], axis=1)

    pooled = _mlp(rows, [(W, None, True) for W in mlp_w], 'max',
                  prep=prep, extras=(q,))
    f1r = f1.reshape(nb * m, -1)

    def prep2(xr, f1b):
        return jnp.concatenate([xr, f1b], axis=1)

    out = _mlp(pooled[None], [(W, None, True) for W in mlp2_w], 'none',
               prep=prep2, extras=(f1r,),
               cprep=pooled.shape[-1] + f1r.shape[-1])
    return out.reshape(nb, m, -1)


def _feature_prop(p1, p2, f1, f2, Ws):
    nb, m, _ = p1.shape
    n = p2.shape[1]
    c = f2.shape[-1]
    idx, d = _knn_b(p1, p2, 3)
    idx_f = jnp.transpose(idx + _offs(nb, n), (2, 0, 1)).reshape(-1)
    rows = _gather_rows(f2.reshape(nb * n, c), idx_f).reshape(3, nb * m, c)
    f1r = f1.reshape(nb * m, -1)
    out = _mlp(rows, [(W, None, True) for W in Ws], 'interp3',
               extras=(d.reshape(nb * m, 3), f1r),
               cprep=c + f1r.shape[-1])
    return out.reshape(nb, m, -1)


def kernel(pc1, pc2, feature1, feature2, params):
    P = params
    x1 = pc1.transpose(0, 2, 1)
    x2 = pc2.transpose(0, 2, 1)
    ft1 = feature1.transpose(0, 2, 1)
    ft2 = feature2.transpose(0, 2, 1)

    xyz0 = jnp.concatenate([x1, x2], 0)
    feat0 = jnp.concatenate([ft1, ft2], 0)

    l0p, l0f = _sa(xyz0, feat0, 2048, 16, P['sa0'])
    l1p, l1f = _sa(l0p, l0f, 2048, 16, P['sa1'])
    l2p, l2f = _sa(l1p, l1f, 512, 16, P['sa2'])

    l2p1, l2p2 = l2p[:2], l2p[2:]
    l2f1, l2f2 = l2f[:2], l2f[2:]
    l1p1, l1f1 = l1p[:2], l1f[:2]

    l2f1n = _flow_embedding(l2p1, l2p2, l2f1, l2f2, 64, P['fe'])

    l3p1, l3f1 = _sa(l2p1, l2f1n, 128, 8, P['sa3'])
    l4p1, l4f1 = _sa(l3p1, l3f1, 32, 8, P['sa4'])

    l3fn = _set_upconv(l3p1, l4p1, l3f1, l4f1, 8, [], P['su1_mlp2'])
    l2fn = _set_upconv(l2p1, l3p1,
                       jnp.concatenate([l2f1, l2f1n], -1), l3fn, 8,
                       P['su2_mlp'], P['su2_mlp2'])
    l1fn = _set_upconv(l1p1, l2p1, l1f1, l2fn, 8, P['su3_mlp'], P['su3_mlp2'])

    l0fn = _feature_prop(x1, l1p1, ft1, l1fn, P['fp'])

    out = _mlp(l0fn.reshape(1, 2 * 8192, 256),
               [(P['conv1'], None, True),
                (P['conv2_w'], P['conv2_b'], False)], 'none')
    return out.reshape(2, 8192, 3)


# speedup vs baseline: 1.7234x; 1.0008x over previous
import functools

import jax
import jax.numpy as jnp
from jax import lax
from jax.experimental import pallas as pl
from jax.experimental.pallas import tpu as pltpu
from jax.experimental.pallas import tpu_sc as plsc

_BIG = float(3.0e38)


def _fps_b(xyz, npoint):
    nb, n, _ = xyz.shape
    cols = 128
    rows = max(1, -(-n // cols))
    rows8 = -(-rows // 8) * 8
    total = rows8 * cols
    pad = total - n
    if pad:
        xyz_p = jnp.concatenate(
            [xyz, jnp.broadcast_to(xyz[:, 0:1, :], (nb, pad, 3))], axis=1)
    else:
        xyz_p = xyz

    planes3 = jnp.transpose(xyz_p, (2, 0, 1)).reshape(3, nb, rows8, cols)

    def body(planes_ref, rows_ref, *rest):
        out_refs = rest[:nb]
        dists_ref = rest[nb]
        r_iota = lax.broadcasted_iota(jnp.int32, (nb, rows8, cols), 1)
        c_iota = lax.broadcasted_iota(jnp.int32, (nb, rows8, cols), 2)
        flat = r_iota * cols + c_iota
        dists_ref[...] = jnp.full((nb, rows8, cols), 1e10, jnp.float32)

        def step(j, fars):
            crows = [rows_ref[c, pl.ds(fars[c], 1), :] for c in range(nb)]
            for c in range(nb):
                out_refs[c][0, j] = fars[c]
            cxyz = jnp.concatenate(crows, axis=0)

            def cplane(a):
                return jnp.broadcast_to(cxyz[:, a][:, None, None],
                                        (nb, rows8, cols))

            dx = planes_ref[0] - cplane(0)
            dy = planes_ref[1] - cplane(1)
            dz = planes_ref[2] - cplane(2)
            d = dx * dx + dy * dy + dz * dz
            nd = jnp.minimum(dists_ref[...], d)
            dists_ref[...] = nd
            mx = jnp.max(nd, axis=(1, 2), keepdims=True)
            fidx = jnp.min(jnp.where(nd == mx, flat, total),
                           axis=(1, 2))
            nbi = lax.broadcasted_iota(jnp.int32, (nb,), 0)
            return tuple(
                jnp.min(jnp.where(nbi == c, fidx, total)).astype(jnp.int32)
                for c in range(nb))

        lax.fori_loop(0, npoint, step, tuple(jnp.int32(0) for _ in range(nb)))

    outs = pl.pallas_call(
        body,
        in_specs=[
            pl.BlockSpec(memory_space=pltpu.VMEM),
            pl.BlockSpec(memory_space=pltpu.VMEM),
        ],
        out_specs=[pl.BlockSpec(memory_space=pltpu.SMEM)] * nb,
        out_shape=[jax.ShapeDtypeStruct((1, npoint), jnp.int32)] * nb,
        scratch_shapes=[pltpu.VMEM((nb, rows8, cols), jnp.float32)],
    )(planes3, xyz_p)
    return jnp.concatenate(outs, axis=0)


def _knn_b(query, points, k):
    nb, m, _ = query.shape
    n = points.shape[1]
    bm = min(m, max(64, min(256, (8192 * 128) // n)))
    qp = jnp.pad(query, ((0, 0), (0, 0), (0, 5)))
    dt = jnp.pad(points.transpose(0, 2, 1), ((0, 0), (0, 5), (0, 0)))

    def body(q_ref, dt_ref, idx_ref, d_ref):
        q = q_ref[0]
        dtm = dt_ref[0]
        qs = jnp.sum(q * q, axis=1, keepdims=True)
        ps = jnp.sum(dtm * dtm, axis=0, keepdims=True)
        prod = lax.dot_general(q, dtm, (((1,), (0,)), ((), ())),
                               preferred_element_type=jnp.float32)
        cur = (-2.0 * prod + qs) + ps
        lane = lax.broadcasted_iota(jnp.int32, (bm, n), 1)
        idx_cols, d_cols = [], []
        for _ in range(k):
            dmin = jnp.min(cur, axis=1, keepdims=True)
            sel = cur == dmin
            ij = jnp.min(jnp.where(sel, lane, n), axis=1, keepdims=True)
            idx_cols.append(ij)
            d_cols.append(dmin)
            cur = jnp.where(lane == ij, _BIG, cur)
        idx_ref[0] = jnp.concatenate(idx_cols, axis=1)
        d_ref[0] = jnp.concatenate(d_cols, axis=1)

    idx, d = pl.pallas_call(
        body,
        grid=(nb, m // bm),
        in_specs=[
            pl.BlockSpec((1, bm, 8), lambda b, i: (b, i, 0)),
            pl.BlockSpec((1, 8, n), lambda b, i: (b, 0, 0)),
        ],
        out_specs=[
            pl.BlockSpec((1, bm, k), lambda b, i: (b, i, 0)),
            pl.BlockSpec((1, bm, k), lambda b, i: (b, i, 0)),
        ],
        out_shape=[
            jax.ShapeDtypeStruct((nb, m, k), jnp.int32),
            jax.ShapeDtypeStruct((nb, m, k), jnp.float32),
        ],
    )(qp, dt)
    return idx, d


def _sc_gather(table, idx):
    V, D = table.shape
    Bi = idx.shape[0]
    info = plsc.get_sparse_core_info()
    NC, NS = info.num_cores, info.num_subcores
    NW = NC * NS
    b_per_w = Bi // NW
    CH = min(b_per_w, 128)
    n_ch = b_per_w // CH
    mesh = plsc.VectorSubcoreMesh(core_axis_name="c", subcore_axis_name="s")

    @functools.partial(
        pl.kernel, mesh=mesh,
        compiler_params=pltpu.CompilerParams(use_tc_tiling_on_sc=False),
        out_type=jax.ShapeDtypeStruct((Bi, D), jnp.float32),
        scratch_types=[
            pltpu.VMEM((b_per_w,), jnp.int32),
            pltpu.VMEM((CH, D), jnp.float32),
            pltpu.VMEM((CH, D), jnp.float32),
            pltpu.SemaphoreType.DMA,
            pltpu.SemaphoreType.DMA,
            pltpu.SemaphoreType.DMA,
            pltpu.SemaphoreType.DMA,
        ],
    )
    def gk(table_hbm, idx_hbm, out_hbm, idx_v, rv0, rv1, gs0, gs1, ss0, ss1):
        wid = lax.axis_index("s") * NC + lax.axis_index("c")
        base = wid * b_per_w
        pltpu.sync_copy(idx_hbm.at[pl.ds(base, b_per_w)], idx_v)
        bufs, gsems, ssems = (rv0, rv1), (gs0, gs1), (ss0, ss1)

        def gather(i):
            return pltpu.async_copy(
                table_hbm.at[idx_v.at[pl.ds(i * CH, CH)]],
                bufs[i % 2], gsems[i % 2])

        def scatter(i):
            return pltpu.async_copy(
                bufs[i % 2], out_hbm.at[pl.ds(base + i * CH, CH)],
                ssems[i % 2])

        cps, scs = {}, {}
        cps[0] = gather(0)
        for i in range(n_ch):
            if i + 1 < n_ch:
                if i >= 1:
                    scs[i - 1].wait()
                cps[i + 1] = gather(i + 1)
            cps[i].wait()
            scs[i] = scatter(i)
        if n_ch >= 2:
            scs[n_ch - 2].wait()
        scs[n_ch - 1].wait()

    return gk(table, idx)


def _gather_rows(table, idx):
    V, D = table.shape
    Dp = -(-D // 16) * 16
    if Dp != D:
        table = jnp.pad(table, ((0, 0), (0, Dp - D)))
    Bi = idx.shape[0]
    Bp = -(-Bi // 256) * 256
    idx_p = jnp.pad(idx, (0, Bp - Bi)) if Bp != Bi else idx
    rows = _sc_gather(table, idx_p.astype(jnp.int32))
    return rows[:Bi, :D]


def _mlp(x3, layers, pool, prep=None, extras=(), cprep=None):
    k, mp, cin = x3.shape
    cw = cprep if cprep is not None else cin
    cout = layers[-1][0].shape[1] if layers else cw
    gm = min(mp, 512)
    while gm > 8 and k * gm * max(cin, cw, cout) * 4 > 4 * 1024 * 1024:
        gm //= 2
    while mp % gm:
        gm //= 2
    ops = [x3]
    in_specs = [pl.BlockSpec((k, gm, cin), lambda i: (0, i, 0))]
    for e in extras:
        ops.append(e)
        ce = e.shape[1]
        in_specs.append(pl.BlockSpec((gm, ce), lambda i: (i, 0)))
    for (W, b, _r) in layers:
        ops.append(W)
        in_specs.append(pl.BlockSpec(W.shape, lambda i: (0, 0)))
        if b is not None:
            ops.append(b.reshape(1, -1))
            in_specs.append(pl.BlockSpec((1, b.size), lambda i: (0, 0)))
    ne = len(extras)

    def body(*refs):
        x_ref, o_ref = refs[0], refs[-1]
        e_vals = [r[...] for r in refs[1:1 + ne]]
        w_refs = refs[1 + ne:-1]

        def chain(x):
            wi = 0
            for (W, b, relu) in layers:
                x = lax.dot_general(x, w_refs[wi][...],
                                    (((1,), (0,)), ((), ())),
                                    preferred_element_type=jnp.float32)
                wi += 1
                if b is not None:
                    x = x + w_refs[wi][...]
                    wi += 1
                if relu:
                    x = jnp.maximum(x, 0.0)
            return x

        def make_x(j):
            xr = x_ref[j]
            return prep(xr, *e_vals) if prep is not None else xr

        if pool == 'max':
            def jstep(j, acc):
                return jnp.maximum(acc, chain(make_x(j)))
            if k <= 16:
                acc = chain(make_x(0))
                for j in range(1, k):
                    acc = jstep(j, acc)
                o_ref[...] = acc
            else:
                o_ref[...] = lax.fori_loop(
                    0, k, jstep, jnp.full((gm, cout), -_BIG, jnp.float32),
                    unroll=8)
        elif pool == 'interp3':
            d_v, f1_v = e_vals
            dd = jnp.maximum(d_v, 1e-10)
            w = 1.0 / dd
            w = w / jnp.sum(w, axis=1, keepdims=True)

            def wj(j):
                return jnp.broadcast_to(w[:, j:j + 1], (gm, cin))

            xi = (x_ref[0] * wj(0) + x_ref[1] * wj(1)) + x_ref[2] * wj(2)
            o_ref[...] = chain(jnp.concatenate([xi, f1_v], axis=1))
        else:
            o_ref[...] = chain(make_x(0))

    return pl.pallas_call(
        body,
        grid=(mp // gm,),
        in_specs=in_specs,
        out_specs=pl.BlockSpec((gm, cout), lambda i: (i, 0)),
        out_shape=jax.ShapeDtypeStruct((mp, cout), jnp.float32),
    )(*ops)


def _offs(nb, n):
    return (jnp.arange(nb, dtype=jnp.int32) * n)[:, None, None]


def _grouped_rows(points, feats, idx):
    nb, n, _ = points.shape
    c = feats.shape[-1]
    k = idx.shape[-1]
    m = idx.shape[1]
    table = jnp.concatenate([points, feats], -1).reshape(nb * n, 3 + c)
    idx_f = jnp.transpose(idx + _offs(nb, n), (2, 0, 1)).reshape(-1)
    return _gather_rows(table, idx_f).reshape(k, nb * m, 3 + c)


def _sa(xyz, feat, npoint, k, Ws):
    nb, n, _ = xyz.shape
    if npoint < n:
        fidx = _fps_b(xyz, npoint)
        tab = xyz.reshape(nb * n, 3)
        gidx = (fidx + jnp.arange(nb, dtype=jnp.int32)[:, None] * n).reshape(-1)
        new_xyz = _gather_rows(tab, gidx).reshape(nb, npoint, 3)
    else:
        new_xyz = xyz
    idx, _ = _knn_b(new_xyz, xyz, k)
    rows = _grouped_rows(xyz, feat, idx)
    q = new_xyz.reshape(nb * npoint, 3)

    def prep(xr, qb):
        return jnp.concatenate([xr[:, :3] - qb, xr[:, 3:]], axis=1)

    out = _mlp(rows, [(W, None, True) for W in Ws], 'max',
               prep=prep, extras=(q,))
    return new_xyz, out.reshape(nb, npoint, -1)


def _flow_embedding(p1, p2, f1, f2, k, Ws):
    nb, m, _ = p1.shape
    idx, _ = _knn_b(p1, p2, k)
    rows = _grouped_rows(p2, f2, idx)
    q = p1.reshape(nb * m, 3)
    f1r = f1.reshape(nb * m, -1)
    c2 = f2.shape[-1]
    c1 = f1r.shape[-1]

    def prep(xr, qb, f1b):
        return jnp.concatenate([xr[:, 3:], f1b, xr[:, :3] - qb], axis=1)

    out = _mlp(rows, [(W, None, True) for W in Ws], 'max',
               prep=prep, extras=(q, f1r), cprep=c2 + c1 + 3)
    return out.reshape(nb, m, -1)


def _set_upconv(p1, p2, f1, f2, k, mlp_w, mlp2_w):
    nb, m, _ = p1.shape
    idx, _ = _knn_b(p1, p2, k)
    rows = _grouped_rows(p2, f2, idx)
    q = p1.reshape(nb * m, 3)

    def prep(xr, qb):
        return jnp.concatenate([xr[:, 3:], xr[:, :3] - qb], axis=1)

    pooled = _mlp(rows, [(W, None, True) for W in mlp_w], 'max',
                  prep=prep, extras=(q,))
    f1r = f1.reshape(nb * m, -1)

    def prep2(xr, f1b):
        return jnp.concatenate([xr, f1b], axis=1)

    out = _mlp(pooled[None], [(W, None, True) for W in mlp2_w], 'none',
               prep=prep2, extras=(f1r,),
               cprep=pooled.shape[-1] + f1r.shape[-1])
    return out.reshape(nb, m, -1)


def _feature_prop(p1, p2, f1, f2, Ws):
    nb, m, _ = p1.shape
    n = p2.shape[1]
    c = f2.shape[-1]
    idx, d = _knn_b(p1, p2, 3)
    idx_f = jnp.transpose(idx + _offs(nb, n), (2, 0, 1)).reshape(-1)
    rows = _gather_rows(f2.reshape(nb * n, c), idx_f).reshape(3, nb * m, c)
    f1r = f1.reshape(nb * m, -1)
    out = _mlp(rows, [(W, None, True) for W in Ws], 'interp3',
               extras=(d.reshape(nb * m, 3), f1r),
               cprep=c + f1r.shape[-1])
    return out.reshape(nb, m, -1)


def kernel(pc1, pc2, feature1, feature2, params):
    P = params
    x1 = pc1.transpose(0, 2, 1)
    x2 = pc2.transpose(0, 2, 1)
    ft1 = feature1.transpose(0, 2, 1)
    ft2 = feature2.transpose(0, 2, 1)

    xyz0 = jnp.concatenate([x1, x2], 0)
    feat0 = jnp.concatenate([ft1, ft2], 0)

    l0p, l0f = _sa(xyz0, feat0, 2048, 16, P['sa0'])
    l1p, l1f = _sa(l0p, l0f, 2048, 16, P['sa1'])
    l2p, l2f = _sa(l1p, l1f, 512, 16, P['sa2'])

    l2p1, l2p2 = l2p[:2], l2p[2:]
    l2f1, l2f2 = l2f[:2], l2f[2:]
    l1p1, l1f1 = l1p[:2], l1f[:2]

    l2f1n = _flow_embedding(l2p1, l2p2, l2f1, l2f2, 64, P['fe'])

    l3p1, l3f1 = _sa(l2p1, l2f1n, 128, 8, P['sa3'])
    l4p1, l4f1 = _sa(l3p1, l3f1, 32, 8, P['sa4'])

    l3fn = _set_upconv(l3p1, l4p1, l3f1, l4f1, 8, [], P['su1_mlp2'])
    l2fn = _set_upconv(l2p1, l3p1,
                       jnp.concatenate([l2f1, l2f1n], -1), l3fn, 8,
                       P['su2_mlp'], P['su2_mlp2'])
    l1fn = _set_upconv(l1p1, l2p1, l1f1, l2fn, 8, P['su3_mlp'], P['su3_mlp2'])

    l0fn = _feature_prop(x1, l1p1, ft1, l1fn, P['fp'])

    out = _mlp(l0fn.reshape(1, 2 * 8192, 256),
               [(P['conv1'], None, True),
                (P['conv2_w'], P['conv2_b'], False)], 'none')
    return out.reshape(2, 8192, 3)
